# Initial kernel scaffold; baseline (speedup 1.0000x reference)
import jax
import jax.numpy as jnp
from jax.experimental import pallas as pl


def kernel(x_source, x_target, edge_attr_ss, edge_attr_tt, edge_attr_st, edge_attr_ts, src_W1, src_b1, src_W2, src_b2, tgt_W1, tgt_b1, tgt_W2, tgt_b2, conv_Wl, conv_bl, conv_Wr, gate_W1, gate_b1, gate_W2, gate_b2, emlp_W1, emlp_b1, emlp_W2, emlp_b2, emlp_W3, emlp_b3, edge_index_ss, edge_index_tt, edge_index_st, edge_index_ts):
    raise NotImplementedError("write your pallas kernel here")


# speedup vs baseline: 1.7129x; 1.7129x over previous
import functools

import jax
import jax.numpy as jnp
from jax import lax
from jax.experimental import pallas as pl
from jax.experimental.pallas import tpu as pltpu
from jax.experimental.pallas import tpu_sc as plsc

N_NODES = 50000
N_EDGES = 128000
H = 128
ED = 16
OUT = 2

NC = 2
NT = 16
NW = NC * NT

EPT = N_EDGES // NT
EB = 125
NEB = EPT // EB
EPW = N_EDGES // NW
EBW = 80
NEBW = EPW // EBW
N_PAD = 50048
RPT = N_PAD // NT
ZB = 184

def _sc_mesh():
    return plsc.VectorSubcoreMesh(core_axis_name="c", subcore_axis_name="s",
                                  num_cores=NC, num_subcores=NT)


def _fill2d(ref, rows, cols, value):
    def body(i, _):
        for j in range(cols // 16):
            ref[i, pl.ds(j * 16, 16)] = jnp.full((16,), value, jnp.float32)
        return 0
    lax.fori_loop(0, rows, body, 0)


def _sc_counts(cols_ss, cols_tt, cols_st, cols_ts):
    out_t = tuple(jax.ShapeDtypeStruct((N_PAD, 16), jnp.float32)
                  for _ in range(4))

    @functools.partial(
        pl.kernel, out_type=out_t, mesh=_sc_mesh(),
        compiler_params=pltpu.CompilerParams(use_tc_tiling_on_sc=False),
        scratch_types=[
            pltpu.VMEM((NEB, EB), jnp.int32),
            pltpu.VMEM((EB, 16), jnp.float32),
            pltpu.VMEM_SHARED((N_PAD, 16), jnp.float32),
            pltpu.VMEM_SHARED((N_PAD, 16), jnp.float32),
            pltpu.VMEM((ZB, 16), jnp.float32),
        ],
    )
    def k(c_ss, c_tt, c_st, c_ts, o_ss, o_tt, o_st, o_ts,
          idx_v, ones_v, acc_a, acc_b, zbuf):
        c = lax.axis_index("c")
        s = lax.axis_index("s")
        _fill2d(ones_v, EB, 16, 1.0)
        _fill2d(zbuf, ZB, 16, 0.0)
        def zero(i, _):
            pltpu.sync_copy(zbuf, acc_a.at[pl.ds(s * RPT + i * ZB, ZB)])
            pltpu.sync_copy(zbuf, acc_b.at[pl.ds(s * RPT + i * ZB, ZB)])
            return 0
        lax.fori_loop(0, RPT // ZB, zero, 0)
        plsc.subcore_barrier()

        for cc, (ca, cb, oa, ob) in enumerate(
                ((c_ss, c_ts, o_ss, o_ts), (c_tt, c_st, o_tt, o_st))):
            @pl.when(c == cc)
            def _():
                for src_h, acc in ((ca, acc_a), (cb, acc_b)):
                    pltpu.sync_copy(src_h.at[s], idx_v)
                    def body(j, _):
                        pltpu.sync_copy(ones_v, acc.at[idx_v.at[j]], add=True)
                        return 0
                    lax.fori_loop(0, NEB, body, 0)
        plsc.subcore_barrier()
        for cc, (oa, ob) in enumerate(((o_ss, o_ts), (o_tt, o_st))):
            @pl.when(c == cc)
            def _():
                pltpu.sync_copy(acc_a.at[pl.ds(s * RPT, RPT)],
                                oa.at[pl.ds(s * RPT, RPT)])
                pltpu.sync_copy(acc_b.at[pl.ds(s * RPT, RPT)],
                                ob.at[pl.ds(s * RPT, RPT)])

    return k(cols_ss, cols_tt, cols_st, cols_ts)


def _sc_layer_sums(xs_chunks, xt_chunks, ridx, cidx):
    out_t = tuple(jax.ShapeDtypeStruct((N_PAD, 32), jnp.float32)
                  for _ in range(16))

    @functools.partial(
        pl.kernel, out_type=out_t, mesh=_sc_mesh(),
        compiler_params=pltpu.CompilerParams(use_tc_tiling_on_sc=False),
        scratch_types=[
            pltpu.VMEM((NEB, EB), jnp.int32),
            pltpu.VMEM((NEB, EB), jnp.int32),
            pltpu.VMEM((EB, 32), jnp.float32),
            pltpu.VMEM_SHARED((N_PAD, 32), jnp.float32),
            pltpu.VMEM((ZB, 32), jnp.float32),
            pltpu.SemaphoreType.DMA,
        ],
    )
    def k(xs0, xs1, xs2, xs3, xt0, xt1, xt2, xt3,
          r_ss, c_ss, r_tt, c_tt, r_st, c_st, r_ts, c_ts,
          *rest):
        outs = rest[:16]
        row_v, col_v, gbuf, acc, zbuf, sem = rest[16:]
        c = lax.axis_index("c")
        s = lax.axis_index("s")
        _fill2d(zbuf, ZB, 32, 0.0)
        xs_t = (xs0, xs1, xs2, xs3)
        xt_t = (xt0, xt1, xt2, xt3)
        cfg = ((xs_t, r_ss, c_ss, outs[0:4]),
               (xt_t, r_tt, c_tt, outs[4:8]),
               (xs_t, r_st, c_st, outs[8:12]),
               (xt_t, r_ts, c_ts, outs[12:16]))
        for cc in range(NC):
            @pl.when(c == cc)
            def _():
                for tbls, r_h, c_h, o_chunks in cfg:
                    pltpu.sync_copy(r_h.at[s], row_v)
                    pltpu.sync_copy(c_h.at[s], col_v)
                    for kk in (2 * cc, 2 * cc + 1):
                        tbl = tbls[kk]
                        def zero(i, _):
                            pltpu.sync_copy(
                                zbuf, acc.at[pl.ds(s * RPT + i * ZB, ZB)])
                            return 0
                        lax.fori_loop(0, RPT // ZB, zero, 0)
                        plsc.subcore_barrier()
                        def body(j, _):
                            pltpu.async_copy(
                                tbl.at[row_v.at[j]], gbuf, sem).wait()
                            pltpu.sync_copy(gbuf, acc.at[col_v.at[j]],
                                            add=True)
                            return 0
                        lax.fori_loop(0, NEB, body, 0)
                        plsc.subcore_barrier()
                        pltpu.sync_copy(
                            acc.at[pl.ds(s * RPT, RPT)],
                            o_chunks[kk].at[pl.ds(s * RPT, RPT)])
                        plsc.subcore_barrier()

    outs = k(*xs_chunks, *xt_chunks,
             ridx["ss"], cidx["ss"], ridx["tt"], cidx["tt"],
             ridx["st"], cidx["st"], ridx["ts"], cidx["ts"])
    return (tuple(outs[0:4]), tuple(outs[4:8]),
            tuple(outs[8:12]), tuple(outs[12:16]))


def _sc_edge_gather(x3s, x3t, rows_w, cols_w):
    out_t = (jax.ShapeDtypeStruct((N_EDGES, H), jnp.float32),
             jax.ShapeDtypeStruct((N_EDGES, H), jnp.float32))

    @functools.partial(
        pl.kernel, out_type=out_t, mesh=_sc_mesh(),
        compiler_params=pltpu.CompilerParams(use_tc_tiling_on_sc=False),
        scratch_types=[
            pltpu.VMEM((NEBW, EBW), jnp.int32),
            pltpu.VMEM((EBW, H), jnp.float32),
            pltpu.SemaphoreType.DMA,
        ],
    )
    def k(xs_h, xt_h, r_h, c_h, o_s, o_t, idx_v, gbuf, sem):
        c = lax.axis_index("c")
        s = lax.axis_index("s")
        wid = s * NC + c
        base = wid * EPW
        for tbl, i_h, o_h in ((xs_h, r_h, o_s), (xt_h, c_h, o_t)):
            pltpu.sync_copy(i_h.at[wid], idx_v)
            def body(j, _):
                pltpu.async_copy(tbl.at[idx_v.at[j]], gbuf, sem).wait()
                pltpu.sync_copy(gbuf, o_h.at[pl.ds(base + j * EBW, EBW)])
                return 0
            lax.fori_loop(0, NEBW, body, 0)

    return k(x3s, x3t, rows_w, cols_w)


BN = 1000
BE = 1000


def _dot(a, b):
    return jax.lax.dot_general(a, b, (((1,), (0,)), ((), ())),
                               preferred_element_type=jnp.float32)


def _chunk_specs(bn):
    return [pl.BlockSpec((bn, 32), lambda i: (i, 0)) for _ in range(4)]


def _write_chunks(o, refs):
    for q, r in enumerate(refs):
        r[...] = o[:, 32 * q:32 * (q + 1)]


def _enc_body(x_ref, w1_ref, b1_ref, w2_ref, b2_ref,
              of_ref, c0, c1, c2, c3):
    h = jnp.maximum(_dot(x_ref[...], w1_ref[...]) + b1_ref[...], 0.0)
    o = _dot(h, w2_ref[...]) + b2_ref[...]
    of_ref[...] = o
    _write_chunks(o, (c0, c1, c2, c3))


def _encoder(x, w1t, b1, w2t, b2):
    n = x.shape[0]
    grid = (n // BN,)
    wspec = pl.BlockSpec((H, H), lambda i: (0, 0))
    bspec = pl.BlockSpec((1, H), lambda i: (0, 0))
    out_shape = ([jax.ShapeDtypeStruct((n, H), jnp.float32)]
                 + [jax.ShapeDtypeStruct((n, 32), jnp.float32)] * 4)
    outs = pl.pallas_call(
        _enc_body,
        grid=grid,
        in_specs=[pl.BlockSpec((BN, H), lambda i: (i, 0)),
                  wspec, bspec, wspec, bspec],
        out_specs=[pl.BlockSpec((BN, H), lambda i: (i, 0))]
        + _chunk_specs(BN),
        out_shape=out_shape,
    )(x, w1t, b1, w2t, b2)
    return outs[0], tuple(outs[1:])


def _comb_body(sa0, sa1, sa2, sa3, ca_ref, sb0, sb1, sb2, sb3, cb_ref,
               x_ref, wa_ref, wb_ref, wc_ref, b_ref, *out_refs,
               residual, chunks):
    inva = 1.0 / jnp.maximum(ca_ref[...][:, 0:1], 1.0)
    invb = 1.0 / jnp.maximum(cb_ref[...][:, 0:1], 1.0)
    wa = wa_ref[...]
    wb = wb_ref[...]
    da = sum(_dot(s[...], wa[32 * q:32 * (q + 1), :])
             for q, s in enumerate((sa0, sa1, sa2, sa3)))
    db = sum(_dot(s[...], wb[32 * q:32 * (q + 1), :])
             for q, s in enumerate((sb0, sb1, sb2, sb3)))
    acc = da * inva + db * invb + _dot(x_ref[...], wc_ref[...])
    o = 0.5 * acc + b_ref[...]
    if residual:
        o = o + x_ref[...]
    o = jnp.maximum(o, 0.0)
    out_refs[0][...] = o
    if chunks:
        _write_chunks(o, out_refs[1:])


def _combine(sums_a, cnt_a, sums_b, cnt_b, x, wa, wb, wc, b,
             residual, chunks):
    n = x.shape[0]
    grid = (n // BN,)
    nspec = pl.BlockSpec((BN, H), lambda i: (i, 0))
    cntspec = pl.BlockSpec((BN, 16), lambda i: (i, 0))
    wspec = pl.BlockSpec((H, H), lambda i: (0, 0))
    bspec = pl.BlockSpec((1, H), lambda i: (0, 0))
    out_shape = [jax.ShapeDtypeStruct((n, H), jnp.float32)]
    out_specs = [nspec]
    if chunks:
        out_shape += [jax.ShapeDtypeStruct((n, 32), jnp.float32)] * 4
        out_specs += _chunk_specs(BN)
    outs = pl.pallas_call(
        functools.partial(_comb_body, residual=residual, chunks=chunks),
        grid=grid,
        in_specs=_chunk_specs(BN) + [cntspec] + _chunk_specs(BN)
        + [cntspec, nspec, wspec, wspec, wspec, bspec],
        out_specs=out_specs,
        out_shape=out_shape,
    )(*sums_a, cnt_a, *sums_b, cnt_b, x, wa, wb, wc, b)
    if chunks:
        return outs[0], tuple(outs[1:])
    return outs[0]


def _readout_body(sh_ref, th_ref, ea_ref,
                  g1s_ref, g1t_ref, gb1_ref, g2_ref, gb2_ref,
                  w1a_ref, w1b_ref, eb1_ref, w2_ref, eb2_ref,
                  w3_ref, eb3_ref, o_ref):
    sh = sh_ref[...]
    th = th_ref[...]
    h1 = jnp.maximum(_dot(sh, g1s_ref[...]) + _dot(th, g1t_ref[...])
                     + gb1_ref[...], 0.0)
    gate = jax.nn.sigmoid(_dot(h1, g2_ref[...]) + gb2_ref[...])
    comb = sh * gate + th * (1.0 - gate)
    h2 = jnp.maximum(_dot(comb, w1a_ref[...]) + _dot(ea_ref[...], w1b_ref[...])
                     + eb1_ref[...], 0.0)
    h3 = jnp.maximum(_dot(h2, w2_ref[...]) + eb2_ref[...], 0.0)
    o_ref[...] = _dot(h3, w3_ref[...]) + eb3_ref[...]


def _readout(src_h, tgt_h, edge_attr, g1s, g1t, gb1, g2, gb2,
             w1a, w1b, eb1, w2, eb2, w3, eb3):
    grid = (N_EDGES // BE,)
    espec = pl.BlockSpec((BE, H), lambda i: (i, 0))

    def c(shape):
        return pl.BlockSpec(shape, lambda i: (0, 0))

    return pl.pallas_call(
        _readout_body,
        grid=grid,
        in_specs=[espec, espec, pl.BlockSpec((BE, ED), lambda i: (i, 0)),
                  c((H, H)), c((H, H)), c((1, H)), c((H, 1)), c((1, 1)),
                  c((H, H)), c((ED, H)), c((1, H)), c((H, H // 2)),
                  c((1, H // 2)), c((H // 2, OUT)), c((1, OUT))],
        out_specs=pl.BlockSpec((BE, OUT), lambda i: (i, 0)),
        out_shape=jax.ShapeDtypeStruct((N_EDGES, OUT), jnp.float32),
    )(src_h, tgt_h, edge_attr, g1s, g1t, gb1, g2, gb2,
      w1a, w1b, eb1, w2, eb2, w3, eb3)


def kernel(x_source, x_target, edge_attr_ss, edge_attr_tt, edge_attr_st,
           edge_attr_ts, src_W1, src_b1, src_W2, src_b2, tgt_W1, tgt_b1,
           tgt_W2, tgt_b2, conv_Wl, conv_bl, conv_Wr, gate_W1, gate_b1,
           gate_W2, gate_b2, emlp_W1, emlp_b1, emlp_W2, emlp_b2, emlp_W3,
           emlp_b3, edge_index_ss, edge_index_tt, edge_index_st,
           edge_index_ts):
    f32 = jnp.float32
    ei = {"ss": edge_index_ss, "tt": edge_index_tt,
          "st": edge_index_st, "ts": edge_index_ts}
    ridx = {k: v[0].astype(jnp.int32).reshape(NT, NEB, EB)
            for k, v in ei.items()}
    cidx = {k: v[1].astype(jnp.int32).reshape(NT, NEB, EB)
            for k, v in ei.items()}

    cnt_ss, cnt_tt, cnt_st, cnt_ts = _sc_counts(
        cidx["ss"], cidx["tt"], cidx["st"], cidx["ts"])

    hs, hs_ch = _encoder(x_source, src_W1.T, src_b1.reshape(1, H),
                         src_W2.T, src_b2.reshape(1, H))
    ht, ht_ch = _encoder(x_target, tgt_W1.T, tgt_b1.reshape(1, H),
                         tgt_W2.T, tgt_b2.reshape(1, H))

    def layer(l, xs, xs_ch, xt, xt_ch, residual, chunks):
        s_ss, s_tt, s_st, s_ts = _sc_layer_sums(xs_ch, xt_ch, ridx, cidx)
        wl = conv_Wl[l]
        wr = conv_Wr[l]
        bl = conv_bl[l]
        o_s = _combine(s_ss, cnt_ss, s_ts, cnt_ts, xs,
                       wl[0].T, wl[3].T, (wr[0] + wr[3]).T,
                       (0.5 * (bl[0] + bl[3])).reshape(1, H),
                       residual, chunks)
        o_t = _combine(s_tt, cnt_tt, s_st, cnt_st, xt,
                       wl[1].T, wl[2].T, (wr[1] + wr[2]).T,
                       (0.5 * (bl[1] + bl[2])).reshape(1, H),
                       residual, chunks)
        return o_s, o_t

    (x1s, x1s_ch), (x1t, x1t_ch) = layer(0, hs, hs_ch, ht, ht_ch,
                                         residual=False, chunks=True)
    (x2s, x2s_ch), (x2t, x2t_ch) = layer(1, x1s, x1s_ch, x1t, x1t_ch,
                                         residual=True, chunks=True)
    x3s, x3t = layer(2, x2s, x2s_ch, x2t, x2t_ch,
                     residual=True, chunks=False)

    rows_w = edge_index_st[0].astype(jnp.int32).reshape(NW, NEBW, EBW)
    cols_w = edge_index_st[1].astype(jnp.int32).reshape(NW, NEBW, EBW)
    src_h, tgt_h = _sc_edge_gather(x3s, x3t, rows_w, cols_w)

    return _readout(
        src_h, tgt_h, edge_attr_st,
        gate_W1[:, :H].T, gate_W1[:, H:].T, gate_b1.reshape(1, H),
        gate_W2.T, gate_b2.reshape(1, 1),
        emlp_W1[:, :H].T, emlp_W1[:, H:].T, emlp_b1.reshape(1, H),
        emlp_W2.T, emlp_b2.reshape(1, H // 2),
        emlp_W3.T, emlp_b3.reshape(1, OUT))


# speedup vs baseline: 2.1841x; 1.2750x over previous
import functools

import jax
import jax.numpy as jnp
from jax import lax
from jax.experimental import pallas as pl
from jax.experimental.pallas import tpu as pltpu
from jax.experimental.pallas import tpu_sc as plsc

N_NODES = 50000
N_EDGES = 128000
H = 128
ED = 16
OUT = 2

NC = 2
NT = 16
NW = NC * NT

EPT = N_EDGES // NT
EB = 125
NEB = EPT // EB
EPW = N_EDGES // NW
EBW = 80
NEBW = EPW // EBW
N_PAD = 50048
RPT = N_PAD // NT
ZB = 184

def _sc_mesh():
    return plsc.VectorSubcoreMesh(core_axis_name="c", subcore_axis_name="s",
                                  num_cores=NC, num_subcores=NT)


def _fill2d(ref, rows, cols, value):
    def body(i, _):
        for j in range(cols // 16):
            ref[i, pl.ds(j * 16, 16)] = jnp.full((16,), value, jnp.float32)
        return 0
    lax.fori_loop(0, rows, body, 0)


def _sc_counts(cols_ss, cols_tt, cols_st, cols_ts):
    out_t = tuple(jax.ShapeDtypeStruct((N_PAD, 16), jnp.float32)
                  for _ in range(4))

    @functools.partial(
        pl.kernel, out_type=out_t, mesh=_sc_mesh(),
        compiler_params=pltpu.CompilerParams(use_tc_tiling_on_sc=False),
        scratch_types=[
            pltpu.VMEM((NEB, EB), jnp.int32),
            pltpu.VMEM((EB, 16), jnp.float32),
            pltpu.VMEM_SHARED((N_PAD, 16), jnp.float32),
            pltpu.VMEM_SHARED((N_PAD, 16), jnp.float32),
            pltpu.VMEM((ZB, 16), jnp.float32),
        ],
    )
    def k(c_ss, c_tt, c_st, c_ts, o_ss, o_tt, o_st, o_ts,
          idx_v, ones_v, acc_a, acc_b, zbuf):
        c = lax.axis_index("c")
        s = lax.axis_index("s")
        _fill2d(ones_v, EB, 16, 1.0)
        _fill2d(zbuf, ZB, 16, 0.0)
        def zero(i, _):
            pltpu.sync_copy(zbuf, acc_a.at[pl.ds(s * RPT + i * ZB, ZB)])
            pltpu.sync_copy(zbuf, acc_b.at[pl.ds(s * RPT + i * ZB, ZB)])
            return 0
        lax.fori_loop(0, RPT // ZB, zero, 0)
        plsc.subcore_barrier()

        for cc, (ca, cb, oa, ob) in enumerate(
                ((c_ss, c_ts, o_ss, o_ts), (c_tt, c_st, o_tt, o_st))):
            @pl.when(c == cc)
            def _():
                for src_h, acc in ((ca, acc_a), (cb, acc_b)):
                    pltpu.sync_copy(src_h.at[s], idx_v)
                    def body(j, _):
                        pltpu.sync_copy(ones_v, acc.at[idx_v.at[j]], add=True)
                        return 0
                    lax.fori_loop(0, NEB, body, 0)
        plsc.subcore_barrier()
        for cc, (oa, ob) in enumerate(((o_ss, o_ts), (o_tt, o_st))):
            @pl.when(c == cc)
            def _():
                pltpu.sync_copy(acc_a.at[pl.ds(s * RPT, RPT)],
                                oa.at[pl.ds(s * RPT, RPT)])
                pltpu.sync_copy(acc_b.at[pl.ds(s * RPT, RPT)],
                                ob.at[pl.ds(s * RPT, RPT)])

    return k(cols_ss, cols_tt, cols_st, cols_ts)


def _sc_layer_sums(xs_chunks, xt_chunks, ridx, cidx):
    out_t = tuple(jax.ShapeDtypeStruct((N_PAD, H), jnp.float32)
                  for _ in range(4))

    @functools.partial(
        pl.kernel, out_type=out_t, mesh=_sc_mesh(),
        compiler_params=pltpu.CompilerParams(use_tc_tiling_on_sc=False),
        scratch_types=[
            pltpu.VMEM((NEB, EB), jnp.int32),
            pltpu.VMEM((NEB, EB), jnp.int32),
            pltpu.VMEM((EB, 32), jnp.float32),
            pltpu.VMEM_SHARED((N_PAD, 32), jnp.float32),
            pltpu.VMEM((ZB, 32), jnp.float32),
            pltpu.SemaphoreType.DMA,
        ],
    )
    def k(xs0, xs1, xs2, xs3, xt0, xt1, xt2, xt3,
          r_ss, c_ss, r_tt, c_tt, r_st, c_st, r_ts, c_ts,
          *rest):
        outs = rest[:4]
        row_v, col_v, gbuf, acc, zbuf, sem = rest[4:]
        c = lax.axis_index("c")
        s = lax.axis_index("s")
        _fill2d(zbuf, ZB, 32, 0.0)
        xs_t = (xs0, xs1, xs2, xs3)
        xt_t = (xt0, xt1, xt2, xt3)
        cfg = ((xs_t, r_ss, c_ss, outs[0]),
               (xt_t, r_tt, c_tt, outs[1]),
               (xs_t, r_st, c_st, outs[2]),
               (xt_t, r_ts, c_ts, outs[3]))
        for cc in range(NC):
            @pl.when(c == cc)
            def _():
                for tbls, r_h, c_h, o_h in cfg:
                    pltpu.sync_copy(r_h.at[s], row_v)
                    pltpu.sync_copy(c_h.at[s], col_v)
                    for kk in (2 * cc, 2 * cc + 1):
                        tbl = tbls[kk]
                        def zero(i, _):
                            pltpu.sync_copy(
                                zbuf, acc.at[pl.ds(s * RPT + i * ZB, ZB)])
                            return 0
                        lax.fori_loop(0, RPT // ZB, zero, 0)
                        plsc.subcore_barrier()
                        def body(j, _):
                            pltpu.async_copy(
                                tbl.at[row_v.at[j]], gbuf, sem).wait()
                            pltpu.sync_copy(gbuf, acc.at[col_v.at[j]],
                                            add=True)
                            return 0
                        lax.fori_loop(0, NEB, body, 0)
                        plsc.subcore_barrier()
                        pltpu.sync_copy(
                            acc.at[pl.ds(s * RPT, RPT)],
                            o_h.at[pl.ds(s * RPT, RPT),
                                   pl.ds(32 * kk, 32)])
                        plsc.subcore_barrier()

    return k(*xs_chunks, *xt_chunks,
             ridx["ss"], cidx["ss"], ridx["tt"], cidx["tt"],
             ridx["st"], cidx["st"], ridx["ts"], cidx["ts"])


def _sc_edge_gather(x3s, x3t, rows_w, cols_w):
    out_t = (jax.ShapeDtypeStruct((N_EDGES, H), jnp.float32),
             jax.ShapeDtypeStruct((N_EDGES, H), jnp.float32))

    @functools.partial(
        pl.kernel, out_type=out_t, mesh=_sc_mesh(),
        compiler_params=pltpu.CompilerParams(use_tc_tiling_on_sc=False),
        scratch_types=[
            pltpu.VMEM((NEBW, EBW), jnp.int32),
            pltpu.VMEM((EBW, H), jnp.float32),
            pltpu.SemaphoreType.DMA,
        ],
    )
    def k(xs_h, xt_h, r_h, c_h, o_s, o_t, idx_v, gbuf, sem):
        c = lax.axis_index("c")
        s = lax.axis_index("s")
        wid = s * NC + c
        base = wid * EPW
        for tbl, i_h, o_h in ((xs_h, r_h, o_s), (xt_h, c_h, o_t)):
            pltpu.sync_copy(i_h.at[wid], idx_v)
            def body(j, _):
                pltpu.async_copy(tbl.at[idx_v.at[j]], gbuf, sem).wait()
                pltpu.sync_copy(gbuf, o_h.at[pl.ds(base + j * EBW, EBW)])
                return 0
            lax.fori_loop(0, NEBW, body, 0)

    return k(x3s, x3t, rows_w, cols_w)


BN = 2000
BE = 1000


def _dot(a, b):
    return jax.lax.dot_general(a, b, (((1,), (0,)), ((), ())),
                               preferred_element_type=jnp.float32)


def _chunk_specs(bn):
    return [pl.BlockSpec((bn, 32), lambda i: (i, 0)) for _ in range(4)]


def _write_chunks(o, refs):
    for q, r in enumerate(refs):
        r[...] = o[:, 32 * q:32 * (q + 1)]


def _enc_body(x_ref, w1_ref, b1_ref, w2_ref, b2_ref,
              of_ref, c0, c1, c2, c3):
    h = jnp.maximum(_dot(x_ref[...], w1_ref[...]) + b1_ref[...], 0.0)
    o = _dot(h, w2_ref[...]) + b2_ref[...]
    of_ref[...] = o
    _write_chunks(o, (c0, c1, c2, c3))


def _encoder(x, w1t, b1, w2t, b2):
    n = x.shape[0]
    grid = (n // BN,)
    wspec = pl.BlockSpec((H, H), lambda i: (0, 0))
    bspec = pl.BlockSpec((1, H), lambda i: (0, 0))
    out_shape = ([jax.ShapeDtypeStruct((n, H), jnp.float32)]
                 + [jax.ShapeDtypeStruct((n, 32), jnp.float32)] * 4)
    outs = pl.pallas_call(
        _enc_body,
        grid=grid,
        in_specs=[pl.BlockSpec((BN, H), lambda i: (i, 0)),
                  wspec, bspec, wspec, bspec],
        out_specs=[pl.BlockSpec((BN, H), lambda i: (i, 0))]
        + _chunk_specs(BN),
        out_shape=out_shape,
    )(x, w1t, b1, w2t, b2)
    return outs[0], tuple(outs[1:])


def _comb_body(sa_ref, ca_ref, sb_ref, cb_ref,
               x_ref, wa_ref, wb_ref, wc_ref, b_ref, *out_refs,
               residual, chunks):
    inva = 1.0 / jnp.maximum(ca_ref[...][:, 0:1], 1.0)
    invb = 1.0 / jnp.maximum(cb_ref[...][:, 0:1], 1.0)
    acc = (_dot(sa_ref[...], wa_ref[...]) * inva
           + _dot(sb_ref[...], wb_ref[...]) * invb
           + _dot(x_ref[...], wc_ref[...]))
    o = 0.5 * acc + b_ref[...]
    if residual:
        o = o + x_ref[...]
    o = jnp.maximum(o, 0.0)
    out_refs[0][...] = o
    if chunks:
        _write_chunks(o, out_refs[1:])


def _combine(sums_a, cnt_a, sums_b, cnt_b, x, wa, wb, wc, b,
             residual, chunks):
    n = x.shape[0]
    grid = (n // BN,)
    nspec = pl.BlockSpec((BN, H), lambda i: (i, 0))
    cntspec = pl.BlockSpec((BN, 16), lambda i: (i, 0))
    wspec = pl.BlockSpec((H, H), lambda i: (0, 0))
    bspec = pl.BlockSpec((1, H), lambda i: (0, 0))
    out_shape = [jax.ShapeDtypeStruct((n, H), jnp.float32)]
    out_specs = [nspec]
    if chunks:
        out_shape += [jax.ShapeDtypeStruct((n, 32), jnp.float32)] * 4
        out_specs += _chunk_specs(BN)
    outs = pl.pallas_call(
        functools.partial(_comb_body, residual=residual, chunks=chunks),
        grid=grid,
        in_specs=[nspec, cntspec, nspec, cntspec, nspec,
                  wspec, wspec, wspec, bspec],
        out_specs=out_specs,
        out_shape=out_shape,
    )(sums_a, cnt_a, sums_b, cnt_b, x, wa, wb, wc, b)
    if chunks:
        return outs[0], tuple(outs[1:])
    return outs[0]


def _readout_body(sh_ref, th_ref, ea_ref,
                  g1s_ref, g1t_ref, gb1_ref, g2_ref, gb2_ref,
                  w1a_ref, w1b_ref, eb1_ref, w2_ref, eb2_ref,
                  w3_ref, eb3_ref, o_ref):
    sh = sh_ref[...]
    th = th_ref[...]
    h1 = jnp.maximum(_dot(sh, g1s_ref[...]) + _dot(th, g1t_ref[...])
                     + gb1_ref[...], 0.0)
    gate = jax.nn.sigmoid(_dot(h1, g2_ref[...]) + gb2_ref[...])
    comb = sh * gate + th * (1.0 - gate)
    h2 = jnp.maximum(_dot(comb, w1a_ref[...]) + _dot(ea_ref[...], w1b_ref[...])
                     + eb1_ref[...], 0.0)
    h3 = jnp.maximum(_dot(h2, w2_ref[...]) + eb2_ref[...], 0.0)
    o_ref[...] = _dot(h3, w3_ref[...]) + eb3_ref[...]


def _readout(src_h, tgt_h, edge_attr, g1s, g1t, gb1, g2, gb2,
             w1a, w1b, eb1, w2, eb2, w3, eb3):
    grid = (N_EDGES // BE,)
    espec = pl.BlockSpec((BE, H), lambda i: (i, 0))

    def c(shape):
        return pl.BlockSpec(shape, lambda i: (0, 0))

    return pl.pallas_call(
        _readout_body,
        grid=grid,
        in_specs=[espec, espec, pl.BlockSpec((BE, ED), lambda i: (i, 0)),
                  c((H, H)), c((H, H)), c((1, H)), c((H, 1)), c((1, 1)),
                  c((H, H)), c((ED, H)), c((1, H)), c((H, H // 2)),
                  c((1, H // 2)), c((H // 2, OUT)), c((1, OUT))],
        out_specs=pl.BlockSpec((BE, OUT), lambda i: (i, 0)),
        out_shape=jax.ShapeDtypeStruct((N_EDGES, OUT), jnp.float32),
    )(src_h, tgt_h, edge_attr, g1s, g1t, gb1, g2, gb2,
      w1a, w1b, eb1, w2, eb2, w3, eb3)


def kernel(x_source, x_target, edge_attr_ss, edge_attr_tt, edge_attr_st,
           edge_attr_ts, src_W1, src_b1, src_W2, src_b2, tgt_W1, tgt_b1,
           tgt_W2, tgt_b2, conv_Wl, conv_bl, conv_Wr, gate_W1, gate_b1,
           gate_W2, gate_b2, emlp_W1, emlp_b1, emlp_W2, emlp_b2, emlp_W3,
           emlp_b3, edge_index_ss, edge_index_tt, edge_index_st,
           edge_index_ts):
    f32 = jnp.float32
    ei = {"ss": edge_index_ss, "tt": edge_index_tt,
          "st": edge_index_st, "ts": edge_index_ts}
    ridx = {k: v[0].astype(jnp.int32).reshape(NT, NEB, EB)
            for k, v in ei.items()}
    cidx = {k: v[1].astype(jnp.int32).reshape(NT, NEB, EB)
            for k, v in ei.items()}

    cnt_ss, cnt_tt, cnt_st, cnt_ts = _sc_counts(
        cidx["ss"], cidx["tt"], cidx["st"], cidx["ts"])

    hs, hs_ch = _encoder(x_source, src_W1.T, src_b1.reshape(1, H),
                         src_W2.T, src_b2.reshape(1, H))
    ht, ht_ch = _encoder(x_target, tgt_W1.T, tgt_b1.reshape(1, H),
                         tgt_W2.T, tgt_b2.reshape(1, H))

    def layer(l, xs, xs_ch, xt, xt_ch, residual, chunks):
        s_ss, s_tt, s_st, s_ts = _sc_layer_sums(xs_ch, xt_ch, ridx, cidx)
        wl = conv_Wl[l]
        wr = conv_Wr[l]
        bl = conv_bl[l]
        o_s = _combine(s_ss, cnt_ss, s_ts, cnt_ts, xs,
                       wl[0].T, wl[3].T, (wr[0] + wr[3]).T,
                       (0.5 * (bl[0] + bl[3])).reshape(1, H),
                       residual, chunks)
        o_t = _combine(s_tt, cnt_tt, s_st, cnt_st, xt,
                       wl[1].T, wl[2].T, (wr[1] + wr[2]).T,
                       (0.5 * (bl[1] + bl[2])).reshape(1, H),
                       residual, chunks)
        return o_s, o_t

    (x1s, x1s_ch), (x1t, x1t_ch) = layer(0, hs, hs_ch, ht, ht_ch,
                                         residual=False, chunks=True)
    (x2s, x2s_ch), (x2t, x2t_ch) = layer(1, x1s, x1s_ch, x1t, x1t_ch,
                                         residual=True, chunks=True)
    x3s, x3t = layer(2, x2s, x2s_ch, x2t, x2t_ch,
                     residual=True, chunks=False)

    rows_w = edge_index_st[0].astype(jnp.int32).reshape(NW, NEBW, EBW)
    cols_w = edge_index_st[1].astype(jnp.int32).reshape(NW, NEBW, EBW)
    src_h, tgt_h = _sc_edge_gather(x3s, x3t, rows_w, cols_w)

    return _readout(
        src_h, tgt_h, edge_attr_st,
        gate_W1[:, :H].T, gate_W1[:, H:].T, gate_b1.reshape(1, H),
        gate_W2.T, gate_b2.reshape(1, 1),
        emlp_W1[:, :H].T, emlp_W1[:, H:].T, emlp_b1.reshape(1, H),
        emlp_W2.T, emlp_b2.reshape(1, H // 2),
        emlp_W3.T, emlp_b3.reshape(1, OUT))


# speedup vs baseline: 2.4013x; 1.0995x over previous
import functools

import jax
import jax.numpy as jnp
from jax import lax
from jax.experimental import pallas as pl
from jax.experimental.pallas import tpu as pltpu
from jax.experimental.pallas import tpu_sc as plsc

N_NODES = 50000
N_EDGES = 128000
H = 128
ED = 16
OUT = 2

NC = 2
NT = 16
NW = NC * NT

EPT = N_EDGES // NT
EB = 125
NEB = EPT // EB
EPW = N_EDGES // NW
EBW = 80
NEBW = EPW // EBW
N_PAD = 50048
RPT = N_PAD // NT
ZB = 184

def _sc_mesh():
    return plsc.VectorSubcoreMesh(core_axis_name="c", subcore_axis_name="s",
                                  num_cores=NC, num_subcores=NT)


def _fill2d(ref, rows, cols, value):
    def body(i, _):
        for j in range(cols // 16):
            ref[i, pl.ds(j * 16, 16)] = jnp.full((16,), value, jnp.float32)
        return 0
    lax.fori_loop(0, rows, body, 0)


def _sc_counts(cols_ss, cols_tt, cols_st, cols_ts):
    out_t = tuple(jax.ShapeDtypeStruct((N_PAD, 16), jnp.float32)
                  for _ in range(4))

    @functools.partial(
        pl.kernel, out_type=out_t, mesh=_sc_mesh(),
        compiler_params=pltpu.CompilerParams(use_tc_tiling_on_sc=False),
        scratch_types=[
            pltpu.VMEM((NEB, EB), jnp.int32),
            pltpu.VMEM((EB, 16), jnp.float32),
            pltpu.VMEM_SHARED((N_PAD, 16), jnp.float32),
            pltpu.VMEM_SHARED((N_PAD, 16), jnp.float32),
            pltpu.VMEM((ZB, 16), jnp.float32),
        ],
    )
    def k(c_ss, c_tt, c_st, c_ts, o_ss, o_tt, o_st, o_ts,
          idx_v, ones_v, acc_a, acc_b, zbuf):
        c = lax.axis_index("c")
        s = lax.axis_index("s")
        _fill2d(ones_v, EB, 16, 1.0)
        _fill2d(zbuf, ZB, 16, 0.0)
        def zero(i, _):
            pltpu.sync_copy(zbuf, acc_a.at[pl.ds(s * RPT + i * ZB, ZB)])
            pltpu.sync_copy(zbuf, acc_b.at[pl.ds(s * RPT + i * ZB, ZB)])
            return 0
        lax.fori_loop(0, RPT // ZB, zero, 0)
        plsc.subcore_barrier()

        for cc, (ca, cb, oa, ob) in enumerate(
                ((c_ss, c_ts, o_ss, o_ts), (c_tt, c_st, o_tt, o_st))):
            @pl.when(c == cc)
            def _():
                for src_h, acc in ((ca, acc_a), (cb, acc_b)):
                    pltpu.sync_copy(src_h.at[s], idx_v)
                    def body(j, _):
                        pltpu.sync_copy(ones_v, acc.at[idx_v.at[j]], add=True)
                        return 0
                    lax.fori_loop(0, NEB, body, 0)
        plsc.subcore_barrier()
        for cc, (oa, ob) in enumerate(((o_ss, o_ts), (o_tt, o_st))):
            @pl.when(c == cc)
            def _():
                pltpu.sync_copy(acc_a.at[pl.ds(s * RPT, RPT)],
                                oa.at[pl.ds(s * RPT, RPT)])
                pltpu.sync_copy(acc_b.at[pl.ds(s * RPT, RPT)],
                                ob.at[pl.ds(s * RPT, RPT)])

    return k(cols_ss, cols_tt, cols_st, cols_ts)


def _sc_layer_sums(xs_chunks, xt_chunks, ridx, cidx):
    out_t = tuple(jax.ShapeDtypeStruct((N_PAD, H), jnp.float32)
                  for _ in range(4))

    @functools.partial(
        pl.kernel, out_type=out_t, mesh=_sc_mesh(),
        compiler_params=pltpu.CompilerParams(use_tc_tiling_on_sc=False),
        scratch_types=[
            pltpu.VMEM((NEB, EB), jnp.int32),
            pltpu.VMEM((NEB, EB), jnp.int32),
            pltpu.VMEM((EB, 32), jnp.float32),
            pltpu.VMEM((EB, 32), jnp.float32),
            pltpu.VMEM_SHARED((N_PAD, 32), jnp.float32),
            pltpu.VMEM((ZB, 32), jnp.float32),
            pltpu.SemaphoreType.DMA,
        ],
    )
    def k(xs0, xs1, xs2, xs3, xt0, xt1, xt2, xt3,
          r_ss, c_ss, r_tt, c_tt, r_st, c_st, r_ts, c_ts,
          *rest):
        outs = rest[:4]
        row_v, col_v, gbuf_a, gbuf_b, acc, zbuf, sem = rest[4:]
        c = lax.axis_index("c")
        s = lax.axis_index("s")
        _fill2d(zbuf, ZB, 32, 0.0)
        xs_t = (xs0, xs1, xs2, xs3)
        xt_t = (xt0, xt1, xt2, xt3)
        cfg = ((xs_t, r_ss, c_ss, outs[0]),
               (xt_t, r_tt, c_tt, outs[1]),
               (xs_t, r_st, c_st, outs[2]),
               (xt_t, r_ts, c_ts, outs[3]))
        for cc in range(NC):
            @pl.when(c == cc)
            def _():
                for tbls, r_h, c_h, o_h in cfg:
                    pltpu.sync_copy(r_h.at[s], row_v)
                    pltpu.sync_copy(c_h.at[s], col_v)
                    for kk in (2 * cc, 2 * cc + 1):
                        tbl = tbls[kk]
                        def zero(i, _):
                            pltpu.sync_copy(
                                zbuf, acc.at[pl.ds(s * RPT + i * ZB, ZB)])
                            return 0
                        lax.fori_loop(0, RPT // ZB, zero, 0)
                        plsc.subcore_barrier()
                        pltpu.async_copy(tbl.at[row_v.at[0]], gbuf_a, sem)
                        def body(j, _):
                            for par, (g_cur, g_nxt) in enumerate(
                                    ((gbuf_a, gbuf_b), (gbuf_b, gbuf_a))):
                                @pl.when(lax.rem(j, 2) == par)
                                def _():
                                    pltpu.make_async_copy(
                                        tbl.at[row_v.at[j]], g_cur,
                                        sem).wait()
                                    @pl.when(j < NEB - 1)
                                    def _():
                                        pltpu.async_copy(
                                            tbl.at[row_v.at[j + 1]],
                                            g_nxt, sem)
                                    pltpu.sync_copy(
                                        g_cur, acc.at[col_v.at[j]],
                                        add=True)
                            return 0
                        lax.fori_loop(0, NEB, body, 0)
                        plsc.subcore_barrier()
                        pltpu.sync_copy(
                            acc.at[pl.ds(s * RPT, RPT)],
                            o_h.at[pl.ds(s * RPT, RPT),
                                   pl.ds(32 * kk, 32)])
                        plsc.subcore_barrier()

    return k(*xs_chunks, *xt_chunks,
             ridx["ss"], cidx["ss"], ridx["tt"], cidx["tt"],
             ridx["st"], cidx["st"], ridx["ts"], cidx["ts"])


def _sc_edge_gather(x3s, x3t, rows_w, cols_w):
    out_t = (jax.ShapeDtypeStruct((N_EDGES, H), jnp.float32),
             jax.ShapeDtypeStruct((N_EDGES, H), jnp.float32))

    @functools.partial(
        pl.kernel, out_type=out_t, mesh=_sc_mesh(),
        compiler_params=pltpu.CompilerParams(use_tc_tiling_on_sc=False),
        scratch_types=[
            pltpu.VMEM((NEBW, EBW), jnp.int32),
            pltpu.VMEM((EBW, H), jnp.float32),
            pltpu.VMEM((EBW, H), jnp.float32),
            pltpu.SemaphoreType.DMA,
        ],
    )
    def k(xs_h, xt_h, r_h, c_h, o_s, o_t, idx_v, gbuf_a, gbuf_b, sem):
        c = lax.axis_index("c")
        s = lax.axis_index("s")
        wid = s * NC + c
        base = wid * EPW
        for tbl, i_h, o_h in ((xs_h, r_h, o_s), (xt_h, c_h, o_t)):
            pltpu.sync_copy(i_h.at[wid], idx_v)
            pltpu.async_copy(tbl.at[idx_v.at[0]], gbuf_a, sem)
            def body(j, _):
                for par, (g_cur, g_nxt) in enumerate(
                        ((gbuf_a, gbuf_b), (gbuf_b, gbuf_a))):
                    @pl.when(lax.rem(j, 2) == par)
                    def _():
                        pltpu.make_async_copy(
                            tbl.at[idx_v.at[j]], g_cur, sem).wait()
                        @pl.when(j < NEBW - 1)
                        def _():
                            pltpu.async_copy(
                                tbl.at[idx_v.at[j + 1]], g_nxt, sem)
                        pltpu.sync_copy(
                            g_cur, o_h.at[pl.ds(base + j * EBW, EBW)])
                return 0
            lax.fori_loop(0, NEBW, body, 0)

    return k(x3s, x3t, rows_w, cols_w)


BN = 2000
BE = 1000


def _dot(a, b):
    return jax.lax.dot_general(a, b, (((1,), (0,)), ((), ())),
                               preferred_element_type=jnp.float32)


def _chunk_specs(bn):
    return [pl.BlockSpec((bn, 32), lambda i: (i, 0)) for _ in range(4)]


def _write_chunks(o, refs):
    for q, r in enumerate(refs):
        r[...] = o[:, 32 * q:32 * (q + 1)]


def _enc_body(x_ref, w1_ref, b1_ref, w2_ref, b2_ref,
              of_ref, c0, c1, c2, c3):
    h = jnp.maximum(_dot(x_ref[...], w1_ref[...]) + b1_ref[...], 0.0)
    o = _dot(h, w2_ref[...]) + b2_ref[...]
    of_ref[...] = o
    _write_chunks(o, (c0, c1, c2, c3))


def _encoder(x, w1t, b1, w2t, b2):
    n = x.shape[0]
    grid = (n // BN,)
    wspec = pl.BlockSpec((H, H), lambda i: (0, 0))
    bspec = pl.BlockSpec((1, H), lambda i: (0, 0))
    out_shape = ([jax.ShapeDtypeStruct((n, H), jnp.float32)]
                 + [jax.ShapeDtypeStruct((n, 32), jnp.float32)] * 4)
    outs = pl.pallas_call(
        _enc_body,
        grid=grid,
        in_specs=[pl.BlockSpec((BN, H), lambda i: (i, 0)),
                  wspec, bspec, wspec, bspec],
        out_specs=[pl.BlockSpec((BN, H), lambda i: (i, 0))]
        + _chunk_specs(BN),
        out_shape=out_shape,
    )(x, w1t, b1, w2t, b2)
    return outs[0], tuple(outs[1:])


def _comb_body(sa_ref, ca_ref, sb_ref, cb_ref,
               x_ref, wa_ref, wb_ref, wc_ref, b_ref, *out_refs,
               residual, chunks):
    inva = 1.0 / jnp.maximum(ca_ref[...][:, 0:1], 1.0)
    invb = 1.0 / jnp.maximum(cb_ref[...][:, 0:1], 1.0)
    acc = (_dot(sa_ref[...], wa_ref[...]) * inva
           + _dot(sb_ref[...], wb_ref[...]) * invb
           + _dot(x_ref[...], wc_ref[...]))
    o = 0.5 * acc + b_ref[...]
    if residual:
        o = o + x_ref[...]
    o = jnp.maximum(o, 0.0)
    out_refs[0][...] = o
    if chunks:
        _write_chunks(o, out_refs[1:])


def _combine(sums_a, cnt_a, sums_b, cnt_b, x, wa, wb, wc, b,
             residual, chunks):
    n = x.shape[0]
    grid = (n // BN,)
    nspec = pl.BlockSpec((BN, H), lambda i: (i, 0))
    cntspec = pl.BlockSpec((BN, 16), lambda i: (i, 0))
    wspec = pl.BlockSpec((H, H), lambda i: (0, 0))
    bspec = pl.BlockSpec((1, H), lambda i: (0, 0))
    out_shape = [jax.ShapeDtypeStruct((n, H), jnp.float32)]
    out_specs = [nspec]
    if chunks:
        out_shape += [jax.ShapeDtypeStruct((n, 32), jnp.float32)] * 4
        out_specs += _chunk_specs(BN)
    outs = pl.pallas_call(
        functools.partial(_comb_body, residual=residual, chunks=chunks),
        grid=grid,
        in_specs=[nspec, cntspec, nspec, cntspec, nspec,
                  wspec, wspec, wspec, bspec],
        out_specs=out_specs,
        out_shape=out_shape,
    )(sums_a, cnt_a, sums_b, cnt_b, x, wa, wb, wc, b)
    if chunks:
        return outs[0], tuple(outs[1:])
    return outs[0]


def _readout_body(sh_ref, th_ref, ea_ref,
                  g1s_ref, g1t_ref, gb1_ref, g2_ref, gb2_ref,
                  w1a_ref, w1b_ref, eb1_ref, w2_ref, eb2_ref,
                  w3_ref, eb3_ref, o_ref):
    sh = sh_ref[...]
    th = th_ref[...]
    h1 = jnp.maximum(_dot(sh, g1s_ref[...]) + _dot(th, g1t_ref[...])
                     + gb1_ref[...], 0.0)
    gate = jax.nn.sigmoid(_dot(h1, g2_ref[...]) + gb2_ref[...])
    comb = sh * gate + th * (1.0 - gate)
    h2 = jnp.maximum(_dot(comb, w1a_ref[...]) + _dot(ea_ref[...], w1b_ref[...])
                     + eb1_ref[...], 0.0)
    h3 = jnp.maximum(_dot(h2, w2_ref[...]) + eb2_ref[...], 0.0)
    o_ref[...] = _dot(h3, w3_ref[...]) + eb3_ref[...]


def _readout(src_h, tgt_h, edge_attr, g1s, g1t, gb1, g2, gb2,
             w1a, w1b, eb1, w2, eb2, w3, eb3):
    grid = (N_EDGES // BE,)
    espec = pl.BlockSpec((BE, H), lambda i: (i, 0))

    def c(shape):
        return pl.BlockSpec(shape, lambda i: (0, 0))

    return pl.pallas_call(
        _readout_body,
        grid=grid,
        in_specs=[espec, espec, pl.BlockSpec((BE, ED), lambda i: (i, 0)),
                  c((H, H)), c((H, H)), c((1, H)), c((H, 1)), c((1, 1)),
                  c((H, H)), c((ED, H)), c((1, H)), c((H, H // 2)),
                  c((1, H // 2)), c((H // 2, OUT)), c((1, OUT))],
        out_specs=pl.BlockSpec((BE, OUT), lambda i: (i, 0)),
        out_shape=jax.ShapeDtypeStruct((N_EDGES, OUT), jnp.float32),
    )(src_h, tgt_h, edge_attr, g1s, g1t, gb1, g2, gb2,
      w1a, w1b, eb1, w2, eb2, w3, eb3)


def kernel(x_source, x_target, edge_attr_ss, edge_attr_tt, edge_attr_st,
           edge_attr_ts, src_W1, src_b1, src_W2, src_b2, tgt_W1, tgt_b1,
           tgt_W2, tgt_b2, conv_Wl, conv_bl, conv_Wr, gate_W1, gate_b1,
           gate_W2, gate_b2, emlp_W1, emlp_b1, emlp_W2, emlp_b2, emlp_W3,
           emlp_b3, edge_index_ss, edge_index_tt, edge_index_st,
           edge_index_ts):
    f32 = jnp.float32
    ei = {"ss": edge_index_ss, "tt": edge_index_tt,
          "st": edge_index_st, "ts": edge_index_ts}
    ridx = {k: v[0].astype(jnp.int32).reshape(NT, NEB, EB)
            for k, v in ei.items()}
    cidx = {k: v[1].astype(jnp.int32).reshape(NT, NEB, EB)
            for k, v in ei.items()}

    cnt_ss, cnt_tt, cnt_st, cnt_ts = _sc_counts(
        cidx["ss"], cidx["tt"], cidx["st"], cidx["ts"])

    hs, hs_ch = _encoder(x_source, src_W1.T, src_b1.reshape(1, H),
                         src_W2.T, src_b2.reshape(1, H))
    ht, ht_ch = _encoder(x_target, tgt_W1.T, tgt_b1.reshape(1, H),
                         tgt_W2.T, tgt_b2.reshape(1, H))

    def layer(l, xs, xs_ch, xt, xt_ch, residual, chunks):
        s_ss, s_tt, s_st, s_ts = _sc_layer_sums(xs_ch, xt_ch, ridx, cidx)
        wl = conv_Wl[l]
        wr = conv_Wr[l]
        bl = conv_bl[l]
        o_s = _combine(s_ss, cnt_ss, s_ts, cnt_ts, xs,
                       wl[0].T, wl[3].T, (wr[0] + wr[3]).T,
                       (0.5 * (bl[0] + bl[3])).reshape(1, H),
                       residual, chunks)
        o_t = _combine(s_tt, cnt_tt, s_st, cnt_st, xt,
                       wl[1].T, wl[2].T, (wr[1] + wr[2]).T,
                       (0.5 * (bl[1] + bl[2])).reshape(1, H),
                       residual, chunks)
        return o_s, o_t

    (x1s, x1s_ch), (x1t, x1t_ch) = layer(0, hs, hs_ch, ht, ht_ch,
                                         residual=False, chunks=True)
    (x2s, x2s_ch), (x2t, x2t_ch) = layer(1, x1s, x1s_ch, x1t, x1t_ch,
                                         residual=True, chunks=True)
    x3s, x3t = layer(2, x2s, x2s_ch, x2t, x2t_ch,
                     residual=True, chunks=False)

    rows_w = edge_index_st[0].astype(jnp.int32).reshape(NW, NEBW, EBW)
    cols_w = edge_index_st[1].astype(jnp.int32).reshape(NW, NEBW, EBW)
    src_h, tgt_h = _sc_edge_gather(x3s, x3t, rows_w, cols_w)

    return _readout(
        src_h, tgt_h, edge_attr_st,
        gate_W1[:, :H].T, gate_W1[:, H:].T, gate_b1.reshape(1, H),
        gate_W2.T, gate_b2.reshape(1, 1),
        emlp_W1[:, :H].T, emlp_W1[:, H:].T, emlp_b1.reshape(1, H),
        emlp_W2.T, emlp_b2.reshape(1, H // 2),
        emlp_W3.T, emlp_b3.reshape(1, OUT))


# speedup vs baseline: 3.7721x; 1.5708x over previous
import functools

import jax
import jax.numpy as jnp
from jax import lax
from jax.experimental import pallas as pl
from jax.experimental.pallas import tpu as pltpu
from jax.experimental.pallas import tpu_sc as plsc

N_NODES = 50000
N_EDGES = 128000
H = 128
ED = 16
OUT = 2

NC = 2
NT = 16
NW = NC * NT

EPT = N_EDGES // NT
EB = 80
NEB = EPT // EB
EPW = N_EDGES // NW
EBW = 80
NEBW = EPW // EBW
N_PAD = 50048
RPT = N_PAD // NT
ZB = 184


def _sc_mesh():
    return plsc.VectorSubcoreMesh(core_axis_name="c", subcore_axis_name="s",
                                  num_cores=NC, num_subcores=NT)


def _fill2d(ref, rows, cols, value):
    def body(i, _):
        for j in range(cols // 16):
            ref[i, pl.ds(j * 16, 16)] = jnp.full((16,), value, jnp.float32)
        return 0
    lax.fori_loop(0, rows, body, 0)


def _sc_counts(cols_ss, cols_tt, cols_st, cols_ts):
    out_t = tuple(jax.ShapeDtypeStruct((N_PAD, 16), jnp.float32)
                  for _ in range(4))

    @functools.partial(
        pl.kernel, out_type=out_t, mesh=_sc_mesh(),
        compiler_params=pltpu.CompilerParams(use_tc_tiling_on_sc=False),
        scratch_types=[
            pltpu.VMEM((NEB, EB), jnp.int32),
            pltpu.VMEM((EB, 16), jnp.float32),
            pltpu.VMEM_SHARED((N_PAD, 16), jnp.float32),
            pltpu.VMEM_SHARED((N_PAD, 16), jnp.float32),
            pltpu.VMEM((ZB, 16), jnp.float32),
        ],
    )
    def k(c_ss, c_tt, c_st, c_ts, o_ss, o_tt, o_st, o_ts,
          idx_v, ones_v, acc_a, acc_b, zbuf):
        c = lax.axis_index("c")
        s = lax.axis_index("s")
        _fill2d(ones_v, EB, 16, 1.0)
        _fill2d(zbuf, ZB, 16, 0.0)
        def zero(i, _):
            pltpu.sync_copy(zbuf, acc_a.at[pl.ds(s * RPT + i * ZB, ZB)])
            pltpu.sync_copy(zbuf, acc_b.at[pl.ds(s * RPT + i * ZB, ZB)])
            return 0
        lax.fori_loop(0, RPT // ZB, zero, 0)
        plsc.subcore_barrier()

        for cc, (ca, cb) in enumerate(((c_ss, c_ts), (c_tt, c_st))):
            @pl.when(c == cc)
            def _():
                for src_h, acc in ((ca, acc_a), (cb, acc_b)):
                    pltpu.sync_copy(src_h.at[s], idx_v)
                    def body(j, _):
                        pltpu.sync_copy(ones_v, acc.at[idx_v.at[j]], add=True)
                        return 0
                    lax.fori_loop(0, NEB, body, 0)
        plsc.subcore_barrier()
        for cc, (oa, ob) in enumerate(((o_ss, o_ts), (o_tt, o_st))):
            @pl.when(c == cc)
            def _():
                pltpu.sync_copy(acc_a.at[pl.ds(s * RPT, RPT)],
                                oa.at[pl.ds(s * RPT, RPT)])
                pltpu.sync_copy(acc_b.at[pl.ds(s * RPT, RPT)],
                                ob.at[pl.ds(s * RPT, RPT)])

    return k(cols_ss, cols_tt, cols_st, cols_ts)


def _sc_layer_sums(xs_tbl, xt_tbl, zrows, ridx, cidx):
    out_t = tuple(jax.ShapeDtypeStruct((N_PAD, H), jnp.float32)
                  for _ in range(4))

    @functools.partial(
        pl.kernel, out_type=out_t, mesh=_sc_mesh(),
        compiler_params=pltpu.CompilerParams(use_tc_tiling_on_sc=False),
        scratch_types=[
            pltpu.VMEM((NEB, EB), jnp.int32),
            pltpu.VMEM((NEB, EB), jnp.int32),
            pltpu.VMEM((EB, 32), jnp.float32),
            pltpu.VMEM((EB, 32), jnp.float32),
            pltpu.VMEM((EB, 32), jnp.float32),
            pltpu.VMEM((EB, 32), jnp.float32),
            pltpu.VMEM_SHARED((N_PAD, 32), jnp.float32),
            pltpu.SemaphoreType.DMA,
            pltpu.SemaphoreType.DMA,
        ],
    )
    def k(xs_h, xt_h, z_h,
          r_ss, c_ss, r_tt, c_tt, r_st, c_st, r_ts, c_ts,
          o_ss, o_tt, o_st, o_ts,
          row_v, col_v, g0, g1, g2, g3, acc, gsem, ssem):
        c = lax.axis_index("c")
        s = lax.axis_index("s")
        gbufs = (g0, g1, g2, g3)
        cfg = ((xs_h, r_ss, c_ss, o_ss),
               (xt_h, r_tt, c_tt, o_tt),
               (xs_h, r_st, c_st, o_st),
               (xt_h, r_ts, c_ts, o_ts))

        def add_inplace(mul, off):
            def body(i, _):
                for u in range(EB // 16):
                    sl = (i, pl.ds(16 * u, 16))
                    row_v[sl] = row_v[sl] * mul + off
                return 0
            lax.fori_loop(0, NEB, body, 0)

        for cc in range(NC):
            @pl.when(c == cc)
            def _():
                for tbl, r_h, c_h, o_h in cfg:
                    pltpu.sync_copy(r_h.at[s], row_v)
                    pltpu.sync_copy(c_h.at[s], col_v)
                    add_inplace(4, 2 * cc)
                    for q, kk in enumerate((2 * cc, 2 * cc + 1)):
                        if q == 1:
                            add_inplace(1, 1)
                        pltpu.sync_copy(z_h, acc.at[pl.ds(s * RPT, RPT)])
                        plsc.subcore_barrier()
                        pltpu.async_copy(tbl.at[row_v.at[0]], g0, gsem)
                        pltpu.async_copy(tbl.at[row_v.at[1]], g1, gsem)
                        pltpu.async_copy(tbl.at[row_v.at[2]], g2, gsem)

                        def body(j, _):
                            for par in range(4):
                                g_cur = gbufs[par]
                                g_pre = gbufs[(par - 1) % 4]
                                g_nxt = gbufs[(par + 3) % 4]

                                @pl.when(lax.rem(j, 4) == par)
                                def _():
                                    pltpu.make_async_copy(
                                        tbl.at[row_v.at[j]], g_cur,
                                        gsem).wait()

                                    @pl.when(j >= 1)
                                    def _():
                                        pltpu.make_async_copy(
                                            g_pre,
                                            acc.at[col_v.at[j - 1]],
                                            ssem).wait()
                                    pltpu.async_copy(
                                        g_cur, acc.at[col_v.at[j]],
                                        ssem, add=True)

                                    @pl.when(j + 3 < NEB)
                                    def _():
                                        pltpu.async_copy(
                                            tbl.at[row_v.at[j + 3]],
                                            g_nxt, gsem)
                            return 0
                        lax.fori_loop(0, NEB, body, 0)
                        pltpu.make_async_copy(
                            gbufs[(NEB - 1) % 4],
                            acc.at[col_v.at[NEB - 1]], ssem).wait()
                        plsc.subcore_barrier()
                        pltpu.sync_copy(
                            acc.at[pl.ds(s * RPT, RPT)],
                            o_h.at[pl.ds(s * RPT, RPT),
                                   pl.ds(32 * kk, 32)])
                        plsc.subcore_barrier()

    return k(xs_tbl, xt_tbl, zrows,
             ridx["ss"], cidx["ss"], ridx["tt"], cidx["tt"],
             ridx["st"], cidx["st"], ridx["ts"], cidx["ts"])


def _sc_edge_gather(x3s, x3t, rows_w, cols_w):
    out_t = (jax.ShapeDtypeStruct((N_EDGES, H), jnp.float32),
             jax.ShapeDtypeStruct((N_EDGES, H), jnp.float32))

    @functools.partial(
        pl.kernel, out_type=out_t, mesh=_sc_mesh(),
        compiler_params=pltpu.CompilerParams(use_tc_tiling_on_sc=False),
        scratch_types=[
            pltpu.VMEM((NEBW, EBW), jnp.int32),
            pltpu.VMEM((EBW, H), jnp.float32),
            pltpu.VMEM((EBW, H), jnp.float32),
            pltpu.SemaphoreType.DMA,
        ],
    )
    def k(xs_h, xt_h, r_h, c_h, o_s, o_t, idx_v, gbuf_a, gbuf_b, sem):
        c = lax.axis_index("c")
        s = lax.axis_index("s")
        wid = s * NC + c
        base = wid * EPW
        for tbl, i_h, o_h in ((xs_h, r_h, o_s), (xt_h, c_h, o_t)):
            pltpu.sync_copy(i_h.at[wid], idx_v)
            pltpu.async_copy(tbl.at[idx_v.at[0]], gbuf_a, sem)
            def body(j, _):
                for par, (g_cur, g_nxt) in enumerate(
                        ((gbuf_a, gbuf_b), (gbuf_b, gbuf_a))):
                    @pl.when(lax.rem(j, 2) == par)
                    def _():
                        pltpu.make_async_copy(
                            tbl.at[idx_v.at[j]], g_cur, sem).wait()
                        @pl.when(j < NEBW - 1)
                        def _():
                            pltpu.async_copy(
                                tbl.at[idx_v.at[j + 1]], g_nxt, sem)
                        pltpu.sync_copy(
                            g_cur, o_h.at[pl.ds(base + j * EBW, EBW)])
                return 0
            lax.fori_loop(0, NEBW, body, 0)

    return k(x3s, x3t, rows_w, cols_w)


BN = 2000
BE = 1000


def _dot(a, b):
    return jax.lax.dot_general(a, b, (((1,), (0,)), ((), ())),
                               preferred_element_type=jnp.float32)


def _enc_body(x_ref, w1_ref, b1_ref, w2_ref, b2_ref, of_ref):
    h = jnp.maximum(_dot(x_ref[...], w1_ref[...]) + b1_ref[...], 0.0)
    of_ref[...] = _dot(h, w2_ref[...]) + b2_ref[...]


def _encoder(x, w1t, b1, w2t, b2):
    n = x.shape[0]
    grid = (n // BN,)
    wspec = pl.BlockSpec((H, H), lambda i: (0, 0))
    bspec = pl.BlockSpec((1, H), lambda i: (0, 0))
    return pl.pallas_call(
        _enc_body,
        grid=grid,
        in_specs=[pl.BlockSpec((BN, H), lambda i: (i, 0)),
                  wspec, bspec, wspec, bspec],
        out_specs=pl.BlockSpec((BN, H), lambda i: (i, 0)),
        out_shape=jax.ShapeDtypeStruct((n, H), jnp.float32),
    )(x, w1t, b1, w2t, b2)


def _comb_body(sa_ref, ca_ref, sb_ref, cb_ref, x_ref,
               wa_ref, wb_ref, wc_ref, b_ref, o_ref, *, residual):
    inva = 1.0 / jnp.maximum(ca_ref[...][:, 0:1], 1.0)
    invb = 1.0 / jnp.maximum(cb_ref[...][:, 0:1], 1.0)
    acc = (_dot(sa_ref[...], wa_ref[...]) * inva
           + _dot(sb_ref[...], wb_ref[...]) * invb
           + _dot(x_ref[...], wc_ref[...]))
    o = 0.5 * acc + b_ref[...]
    if residual:
        o = o + x_ref[...]
    o_ref[...] = jnp.maximum(o, 0.0)


def _combine(sums_a, cnt_a, sums_b, cnt_b, x, wa, wb, wc, b, residual):
    n = x.shape[0]
    grid = (n // BN,)
    nspec = pl.BlockSpec((BN, H), lambda i: (i, 0))
    cntspec = pl.BlockSpec((BN, 16), lambda i: (i, 0))
    wspec = pl.BlockSpec((H, H), lambda i: (0, 0))
    bspec = pl.BlockSpec((1, H), lambda i: (0, 0))
    return pl.pallas_call(
        functools.partial(_comb_body, residual=residual),
        grid=grid,
        in_specs=[nspec, cntspec, nspec, cntspec, nspec,
                  wspec, wspec, wspec, bspec],
        out_specs=nspec,
        out_shape=jax.ShapeDtypeStruct((n, H), jnp.float32),
    )(sums_a, cnt_a, sums_b, cnt_b, x, wa, wb, wc, b)


def _readout_body(sh_ref, th_ref, ea_ref,
                  g1s_ref, g1t_ref, gb1_ref, g2_ref, gb2_ref,
                  w1a_ref, w1b_ref, eb1_ref, w2_ref, eb2_ref,
                  w3_ref, eb3_ref, o_ref):
    sh = sh_ref[...]
    th = th_ref[...]
    h1 = jnp.maximum(_dot(sh, g1s_ref[...]) + _dot(th, g1t_ref[...])
                     + gb1_ref[...], 0.0)
    gate = jax.nn.sigmoid(_dot(h1, g2_ref[...]) + gb2_ref[...])
    comb = sh * gate + th * (1.0 - gate)
    h2 = jnp.maximum(_dot(comb, w1a_ref[...]) + _dot(ea_ref[...], w1b_ref[...])
                     + eb1_ref[...], 0.0)
    h3 = jnp.maximum(_dot(h2, w2_ref[...]) + eb2_ref[...], 0.0)
    o_ref[...] = _dot(h3, w3_ref[...]) + eb3_ref[...]


def _readout(src_h, tgt_h, edge_attr, g1s, g1t, gb1, g2, gb2,
             w1a, w1b, eb1, w2, eb2, w3, eb3):
    grid = (N_EDGES // BE,)
    espec = pl.BlockSpec((BE, H), lambda i: (i, 0))

    def c(shape):
        return pl.BlockSpec(shape, lambda i: (0, 0))

    return pl.pallas_call(
        _readout_body,
        grid=grid,
        in_specs=[espec, espec, pl.BlockSpec((BE, ED), lambda i: (i, 0)),
                  c((H, H)), c((H, H)), c((1, H)), c((H, 1)), c((1, 1)),
                  c((H, H)), c((ED, H)), c((1, H)), c((H, H // 2)),
                  c((1, H // 2)), c((H // 2, OUT)), c((1, OUT))],
        out_specs=pl.BlockSpec((BE, OUT), lambda i: (i, 0)),
        out_shape=jax.ShapeDtypeStruct((N_EDGES, OUT), jnp.float32),
    )(src_h, tgt_h, edge_attr, g1s, g1t, gb1, g2, gb2,
      w1a, w1b, eb1, w2, eb2, w3, eb3)


def kernel(x_source, x_target, edge_attr_ss, edge_attr_tt, edge_attr_st,
           edge_attr_ts, src_W1, src_b1, src_W2, src_b2, tgt_W1, tgt_b1,
           tgt_W2, tgt_b2, conv_Wl, conv_bl, conv_Wr, gate_W1, gate_b1,
           gate_W2, gate_b2, emlp_W1, emlp_b1, emlp_W2, emlp_b2, emlp_W3,
           emlp_b3, edge_index_ss, edge_index_tt, edge_index_st,
           edge_index_ts):
    ei = {"ss": edge_index_ss, "tt": edge_index_tt,
          "st": edge_index_st, "ts": edge_index_ts}
    ridx = {k: v[0].astype(jnp.int32).reshape(NT, NEB, EB)
            for k, v in ei.items()}
    cidx = {k: v[1].astype(jnp.int32).reshape(NT, NEB, EB)
            for k, v in ei.items()}
    zrows = jnp.zeros((RPT, 32), jnp.float32)

    cnt_ss, cnt_tt, cnt_st, cnt_ts = _sc_counts(
        cidx["ss"], cidx["tt"], cidx["st"], cidx["ts"])

    hs = _encoder(x_source, src_W1.T, src_b1.reshape(1, H),
                  src_W2.T, src_b2.reshape(1, H))
    ht = _encoder(x_target, tgt_W1.T, tgt_b1.reshape(1, H),
                  tgt_W2.T, tgt_b2.reshape(1, H))

    def layer(l, xs, xt, residual):
        s_ss, s_tt, s_st, s_ts = _sc_layer_sums(
            xs.reshape(4 * N_NODES, 32), xt.reshape(4 * N_NODES, 32),
            zrows, ridx, cidx)
        wl = conv_Wl[l]
        wr = conv_Wr[l]
        bl = conv_bl[l]
        o_s = _combine(s_ss, cnt_ss, s_ts, cnt_ts, xs,
                       wl[0].T, wl[3].T, (wr[0] + wr[3]).T,
                       (0.5 * (bl[0] + bl[3])).reshape(1, H), residual)
        o_t = _combine(s_tt, cnt_tt, s_st, cnt_st, xt,
                       wl[1].T, wl[2].T, (wr[1] + wr[2]).T,
                       (0.5 * (bl[1] + bl[2])).reshape(1, H), residual)
        return o_s, o_t

    x1s, x1t = layer(0, hs, ht, residual=False)
    x2s, x2t = layer(1, x1s, x1t, residual=True)
    x3s, x3t = layer(2, x2s, x2t, residual=True)

    rows_w = edge_index_st[0].astype(jnp.int32).reshape(NW, NEBW, EBW)
    cols_w = edge_index_st[1].astype(jnp.int32).reshape(NW, NEBW, EBW)
    src_h, tgt_h = _sc_edge_gather(x3s, x3t, rows_w, cols_w)

    return _readout(
        src_h, tgt_h, edge_attr_st,
        gate_W1[:, :H].T, gate_W1[:, H:].T, gate_b1.reshape(1, H),
        gate_W2.T, gate_b2.reshape(1, 1),
        emlp_W1[:, :H].T, emlp_W1[:, H:].T, emlp_b1.reshape(1, H),
        emlp_W2.T, emlp_b2.reshape(1, H // 2),
        emlp_W3.T, emlp_b3.reshape(1, OUT))


# speedup vs baseline: 3.9512x; 1.0475x over previous
import functools

import jax
import jax.numpy as jnp
from jax import lax
from jax.experimental import pallas as pl
from jax.experimental.pallas import tpu as pltpu
from jax.experimental.pallas import tpu_sc as plsc

N_NODES = 50000
N_EDGES = 128000
H = 128
ED = 16
OUT = 2

NC = 2
NT = 16
NW = NC * NT

EPT = N_EDGES // NT
EB = 80
NEB = EPT // EB
EPW = N_EDGES // NW
EBW = 80
NEBW = EPW // EBW
N_PAD = 50048
RPT = N_PAD // NT
ZB = 184


def _sc_mesh():
    return plsc.VectorSubcoreMesh(core_axis_name="c", subcore_axis_name="s",
                                  num_cores=NC, num_subcores=NT)


def _fill2d(ref, rows, cols, value):
    def body(i, _):
        for j in range(cols // 16):
            ref[i, pl.ds(j * 16, 16)] = jnp.full((16,), value, jnp.float32)
        return 0
    lax.fori_loop(0, rows, body, 0)


def _sc_counts(cols_ss, cols_tt, cols_st, cols_ts):
    out_t = tuple(jax.ShapeDtypeStruct((N_PAD, 16), jnp.float32)
                  for _ in range(4))

    @functools.partial(
        pl.kernel, out_type=out_t, mesh=_sc_mesh(),
        compiler_params=pltpu.CompilerParams(use_tc_tiling_on_sc=False),
        scratch_types=[
            pltpu.VMEM((NEB, EB), jnp.int32),
            pltpu.VMEM((EB, 16), jnp.float32),
            pltpu.VMEM_SHARED((N_PAD, 16), jnp.float32),
            pltpu.VMEM_SHARED((N_PAD, 16), jnp.float32),
            pltpu.VMEM((ZB, 16), jnp.float32),
        ],
    )
    def k(c_ss, c_tt, c_st, c_ts, o_ss, o_tt, o_st, o_ts,
          idx_v, ones_v, acc_a, acc_b, zbuf):
        c = lax.axis_index("c")
        s = lax.axis_index("s")
        _fill2d(ones_v, EB, 16, 1.0)
        _fill2d(zbuf, ZB, 16, 0.0)
        def zero(i, _):
            pltpu.sync_copy(zbuf, acc_a.at[pl.ds(s * RPT + i * ZB, ZB)])
            pltpu.sync_copy(zbuf, acc_b.at[pl.ds(s * RPT + i * ZB, ZB)])
            return 0
        lax.fori_loop(0, RPT // ZB, zero, 0)
        plsc.subcore_barrier()

        for cc, (ca, cb) in enumerate(((c_ss, c_ts), (c_tt, c_st))):
            @pl.when(c == cc)
            def _():
                for src_h, acc in ((ca, acc_a), (cb, acc_b)):
                    pltpu.sync_copy(src_h.at[s], idx_v)
                    def body(j, _):
                        pltpu.sync_copy(ones_v, acc.at[idx_v.at[j]], add=True)
                        return 0
                    lax.fori_loop(0, NEB, body, 0)
        plsc.subcore_barrier()
        for cc, (oa, ob) in enumerate(((o_ss, o_ts), (o_tt, o_st))):
            @pl.when(c == cc)
            def _():
                pltpu.sync_copy(acc_a.at[pl.ds(s * RPT, RPT)],
                                oa.at[pl.ds(s * RPT, RPT)])
                pltpu.sync_copy(acc_b.at[pl.ds(s * RPT, RPT)],
                                ob.at[pl.ds(s * RPT, RPT)])

    return k(cols_ss, cols_tt, cols_st, cols_ts)


def _sc_pair_sums(tbl_a, tbl_b, zrows, ra, ca, rb, cb):
    out_t = tuple(jax.ShapeDtypeStruct((N_PAD, H), jnp.float32)
                  for _ in range(2))

    @functools.partial(
        pl.kernel, out_type=out_t, mesh=_sc_mesh(),
        compiler_params=pltpu.CompilerParams(use_tc_tiling_on_sc=False),
        scratch_types=[
            pltpu.VMEM((NEB, EB), jnp.int32),
            pltpu.VMEM((NEB, EB), jnp.int32),
            pltpu.VMEM((EB, 32), jnp.float32),
            pltpu.VMEM((EB, 32), jnp.float32),
            pltpu.VMEM((EB, 32), jnp.float32),
            pltpu.VMEM((EB, 32), jnp.float32),
            pltpu.VMEM_SHARED((N_PAD, 32), jnp.float32),
            pltpu.SemaphoreType.DMA,
            pltpu.SemaphoreType.DMA,
        ],
    )
    def k(ta_h, tb_h, z_h,
          r_a, c_a, r_b, c_b,
          o_a, o_b,
          row_v, col_v, g0, g1, g2, g3, acc, gsem, ssem):
        c = lax.axis_index("c")
        s = lax.axis_index("s")
        gbufs = (g0, g1, g2, g3)
        cfg = ((ta_h, r_a, c_a, o_a),
               (tb_h, r_b, c_b, o_b))

        def add_inplace(mul, off):
            def body(i, _):
                for u in range(EB // 16):
                    sl = (i, pl.ds(16 * u, 16))
                    row_v[sl] = row_v[sl] * mul + off
                return 0
            lax.fori_loop(0, NEB, body, 0)

        for cc in range(NC):
            @pl.when(c == cc)
            def _():
                for tbl, r_h, c_h, o_h in cfg:
                    pltpu.sync_copy(r_h.at[s], row_v)
                    pltpu.sync_copy(c_h.at[s], col_v)
                    add_inplace(4, 2 * cc)
                    for q, kk in enumerate((2 * cc, 2 * cc + 1)):
                        if q == 1:
                            add_inplace(1, 1)
                        pltpu.sync_copy(z_h, acc.at[pl.ds(s * RPT, RPT)])
                        plsc.subcore_barrier()
                        pltpu.async_copy(tbl.at[row_v.at[0]], g0, gsem)
                        pltpu.async_copy(tbl.at[row_v.at[1]], g1, gsem)
                        pltpu.async_copy(tbl.at[row_v.at[2]], g2, gsem)

                        def body(j, _):
                            for par in range(4):
                                g_cur = gbufs[par]
                                g_pre = gbufs[(par - 1) % 4]
                                g_nxt = gbufs[(par + 3) % 4]

                                @pl.when(lax.rem(j, 4) == par)
                                def _():
                                    pltpu.make_async_copy(
                                        tbl.at[row_v.at[j]], g_cur,
                                        gsem).wait()

                                    @pl.when(j >= 1)
                                    def _():
                                        pltpu.make_async_copy(
                                            g_pre,
                                            acc.at[col_v.at[j - 1]],
                                            ssem).wait()
                                    pltpu.async_copy(
                                        g_cur, acc.at[col_v.at[j]],
                                        ssem, add=True)

                                    @pl.when(j + 3 < NEB)
                                    def _():
                                        pltpu.async_copy(
                                            tbl.at[row_v.at[j + 3]],
                                            g_nxt, gsem)
                            return 0
                        lax.fori_loop(0, NEB, body, 0)
                        pltpu.make_async_copy(
                            gbufs[(NEB - 1) % 4],
                            acc.at[col_v.at[NEB - 1]], ssem).wait()
                        plsc.subcore_barrier()
                        pltpu.sync_copy(
                            acc.at[pl.ds(s * RPT, RPT)],
                            o_h.at[pl.ds(s * RPT, RPT),
                                   pl.ds(32 * kk, 32)])
                        plsc.subcore_barrier()

    return k(tbl_a, tbl_b, zrows, ra, ca, rb, cb)


def _sc_edge_gather(x3s, x3t, rows_w, cols_w):
    out_t = (jax.ShapeDtypeStruct((N_EDGES, H), jnp.float32),
             jax.ShapeDtypeStruct((N_EDGES, H), jnp.float32))

    @functools.partial(
        pl.kernel, out_type=out_t, mesh=_sc_mesh(),
        compiler_params=pltpu.CompilerParams(use_tc_tiling_on_sc=False),
        scratch_types=[
            pltpu.VMEM((NEBW, EBW), jnp.int32),
            pltpu.VMEM((EBW, H), jnp.float32),
            pltpu.VMEM((EBW, H), jnp.float32),
            pltpu.SemaphoreType.DMA,
        ],
    )
    def k(xs_h, xt_h, r_h, c_h, o_s, o_t, idx_v, gbuf_a, gbuf_b, sem):
        c = lax.axis_index("c")
        s = lax.axis_index("s")
        wid = s * NC + c
        base = wid * EPW
        for tbl, i_h, o_h in ((xs_h, r_h, o_s), (xt_h, c_h, o_t)):
            pltpu.sync_copy(i_h.at[wid], idx_v)
            pltpu.async_copy(tbl.at[idx_v.at[0]], gbuf_a, sem)
            def body(j, _):
                for par, (g_cur, g_nxt) in enumerate(
                        ((gbuf_a, gbuf_b), (gbuf_b, gbuf_a))):
                    @pl.when(lax.rem(j, 2) == par)
                    def _():
                        pltpu.make_async_copy(
                            tbl.at[idx_v.at[j]], g_cur, sem).wait()
                        @pl.when(j < NEBW - 1)
                        def _():
                            pltpu.async_copy(
                                tbl.at[idx_v.at[j + 1]], g_nxt, sem)
                        pltpu.sync_copy(
                            g_cur, o_h.at[pl.ds(base + j * EBW, EBW)])
                return 0
            lax.fori_loop(0, NEBW, body, 0)

    return k(x3s, x3t, rows_w, cols_w)


BN = 2000
BE = 2000


def _dot(a, b):
    return jax.lax.dot_general(a, b, (((1,), (0,)), ((), ())),
                               preferred_element_type=jnp.float32)


def _enc_body(x_ref, w1_ref, b1_ref, w2_ref, b2_ref, of_ref):
    h = jnp.maximum(_dot(x_ref[...], w1_ref[...]) + b1_ref[...], 0.0)
    of_ref[...] = _dot(h, w2_ref[...]) + b2_ref[...]


def _encoder(x, w1t, b1, w2t, b2):
    n = x.shape[0]
    grid = (n // BN,)
    wspec = pl.BlockSpec((H, H), lambda i: (0, 0))
    bspec = pl.BlockSpec((1, H), lambda i: (0, 0))
    return pl.pallas_call(
        _enc_body,
        grid=grid,
        in_specs=[pl.BlockSpec((BN, H), lambda i: (i, 0)),
                  wspec, bspec, wspec, bspec],
        out_specs=pl.BlockSpec((BN, H), lambda i: (i, 0)),
        out_shape=jax.ShapeDtypeStruct((n, H), jnp.float32),
    )(x, w1t, b1, w2t, b2)


def _comb_body(sa_ref, ca_ref, sb_ref, cb_ref, x_ref,
               wa_ref, wb_ref, wc_ref, b_ref, o_ref, *, residual):
    inva = 1.0 / jnp.maximum(ca_ref[...][:, 0:1], 1.0)
    invb = 1.0 / jnp.maximum(cb_ref[...][:, 0:1], 1.0)
    acc = (_dot(sa_ref[...], wa_ref[...]) * inva
           + _dot(sb_ref[...], wb_ref[...]) * invb
           + _dot(x_ref[...], wc_ref[...]))
    o = 0.5 * acc + b_ref[...]
    if residual:
        o = o + x_ref[...]
    o_ref[...] = jnp.maximum(o, 0.0)


def _combine(sums_a, cnt_a, sums_b, cnt_b, x, wa, wb, wc, b, residual):
    n = x.shape[0]
    grid = (n // BN,)
    nspec = pl.BlockSpec((BN, H), lambda i: (i, 0))
    cntspec = pl.BlockSpec((BN, 16), lambda i: (i, 0))
    wspec = pl.BlockSpec((H, H), lambda i: (0, 0))
    bspec = pl.BlockSpec((1, H), lambda i: (0, 0))
    return pl.pallas_call(
        functools.partial(_comb_body, residual=residual),
        grid=grid,
        in_specs=[nspec, cntspec, nspec, cntspec, nspec,
                  wspec, wspec, wspec, bspec],
        out_specs=nspec,
        out_shape=jax.ShapeDtypeStruct((n, H), jnp.float32),
    )(sums_a, cnt_a, sums_b, cnt_b, x, wa, wb, wc, b)


def _readout_body(sh_ref, th_ref, ea_ref,
                  g1s_ref, g1t_ref, gb1_ref, g2_ref, gb2_ref,
                  w1a_ref, w1b_ref, eb1_ref, w2_ref, eb2_ref,
                  w3_ref, eb3_ref, o_ref):
    sh = sh_ref[...]
    th = th_ref[...]
    h1 = jnp.maximum(_dot(sh, g1s_ref[...]) + _dot(th, g1t_ref[...])
                     + gb1_ref[...], 0.0)
    gate = jax.nn.sigmoid(_dot(h1, g2_ref[...]) + gb2_ref[...])
    comb = sh * gate + th * (1.0 - gate)
    h2 = jnp.maximum(_dot(comb, w1a_ref[...]) + _dot(ea_ref[...], w1b_ref[...])
                     + eb1_ref[...], 0.0)
    h3 = jnp.maximum(_dot(h2, w2_ref[...]) + eb2_ref[...], 0.0)
    o_ref[...] = _dot(h3, w3_ref[...]) + eb3_ref[...]


def _readout(src_h, tgt_h, edge_attr, g1s, g1t, gb1, g2, gb2,
             w1a, w1b, eb1, w2, eb2, w3, eb3):
    grid = (N_EDGES // BE,)
    espec = pl.BlockSpec((BE, H), lambda i: (i, 0))

    def c(shape):
        return pl.BlockSpec(shape, lambda i: (0, 0))

    return pl.pallas_call(
        _readout_body,
        grid=grid,
        in_specs=[espec, espec, pl.BlockSpec((BE, ED), lambda i: (i, 0)),
                  c((H, H)), c((H, H)), c((1, H)), c((H, 1)), c((1, 1)),
                  c((H, H)), c((ED, H)), c((1, H)), c((H, H // 2)),
                  c((1, H // 2)), c((H // 2, OUT)), c((1, OUT))],
        out_specs=pl.BlockSpec((BE, OUT), lambda i: (i, 0)),
        out_shape=jax.ShapeDtypeStruct((N_EDGES, OUT), jnp.float32),
    )(src_h, tgt_h, edge_attr, g1s, g1t, gb1, g2, gb2,
      w1a, w1b, eb1, w2, eb2, w3, eb3)


def kernel(x_source, x_target, edge_attr_ss, edge_attr_tt, edge_attr_st,
           edge_attr_ts, src_W1, src_b1, src_W2, src_b2, tgt_W1, tgt_b1,
           tgt_W2, tgt_b2, conv_Wl, conv_bl, conv_Wr, gate_W1, gate_b1,
           gate_W2, gate_b2, emlp_W1, emlp_b1, emlp_W2, emlp_b2, emlp_W3,
           emlp_b3, edge_index_ss, edge_index_tt, edge_index_st,
           edge_index_ts):
    ei = {"ss": edge_index_ss, "tt": edge_index_tt,
          "st": edge_index_st, "ts": edge_index_ts}
    ridx = {k: v[0].astype(jnp.int32).reshape(NT, NEB, EB)
            for k, v in ei.items()}
    cidx = {k: v[1].astype(jnp.int32).reshape(NT, NEB, EB)
            for k, v in ei.items()}
    zrows = jnp.zeros((RPT, 32), jnp.float32)

    cnt_ss, cnt_tt, cnt_st, cnt_ts = _sc_counts(
        cidx["ss"], cidx["tt"], cidx["st"], cidx["ts"])

    hs = _encoder(x_source, src_W1.T, src_b1.reshape(1, H),
                  src_W2.T, src_b2.reshape(1, H))
    ht = _encoder(x_target, tgt_W1.T, tgt_b1.reshape(1, H),
                  tgt_W2.T, tgt_b2.reshape(1, H))

    def layer(l, xs, xt, residual):
        xs_tbl = xs.reshape(4 * N_NODES, 32)
        xt_tbl = xt.reshape(4 * N_NODES, 32)
        s_ss, s_ts = _sc_pair_sums(xs_tbl, xt_tbl, zrows,
                                   ridx["ss"], cidx["ss"],
                                   ridx["ts"], cidx["ts"])
        s_tt, s_st = _sc_pair_sums(xt_tbl, xs_tbl, zrows,
                                   ridx["tt"], cidx["tt"],
                                   ridx["st"], cidx["st"])
        wl = conv_Wl[l]
        wr = conv_Wr[l]
        bl = conv_bl[l]
        o_s = _combine(s_ss, cnt_ss, s_ts, cnt_ts, xs,
                       wl[0].T, wl[3].T, (wr[0] + wr[3]).T,
                       (0.5 * (bl[0] + bl[3])).reshape(1, H), residual)
        o_t = _combine(s_tt, cnt_tt, s_st, cnt_st, xt,
                       wl[1].T, wl[2].T, (wr[1] + wr[2]).T,
                       (0.5 * (bl[1] + bl[2])).reshape(1, H), residual)
        return o_s, o_t

    x1s, x1t = layer(0, hs, ht, residual=False)
    x2s, x2t = layer(1, x1s, x1t, residual=True)
    x3s, x3t = layer(2, x2s, x2t, residual=True)

    rows_w = edge_index_st[0].astype(jnp.int32).reshape(NW, NEBW, EBW)
    cols_w = edge_index_st[1].astype(jnp.int32).reshape(NW, NEBW, EBW)
    src_h, tgt_h = _sc_edge_gather(x3s, x3t, rows_w, cols_w)

    return _readout(
        src_h, tgt_h, edge_attr_st,
        gate_W1[:, :H].T, gate_W1[:, H:].T, gate_b1.reshape(1, H),
        gate_W2.T, gate_b2.reshape(1, 1),
        emlp_W1[:, :H].T, emlp_W1[:, H:].T, emlp_b1.reshape(1, H),
        emlp_W2.T, emlp_b2.reshape(1, H // 2),
        emlp_W3.T, emlp_b3.reshape(1, OUT))


# speedup vs baseline: 4.2605x; 1.0783x over previous
import functools

import jax
import jax.numpy as jnp
from jax import lax
from jax.experimental import pallas as pl
from jax.experimental.pallas import tpu as pltpu
from jax.experimental.pallas import tpu_sc as plsc

N_NODES = 50000
N_EDGES = 128000
H = 128
ED = 16
OUT = 2

NC = 2
NT = 16
NW = NC * NT

EPT = N_EDGES // NT
EB = 80
NEB = EPT // EB
EPW = N_EDGES // NW
EBW = 80
NEBW = EPW // EBW
N_PAD = 50048
RPT = N_PAD // NT
ZB = 184


def _sc_mesh():
    return plsc.VectorSubcoreMesh(core_axis_name="c", subcore_axis_name="s",
                                  num_cores=NC, num_subcores=NT)


def _fill2d(ref, rows, cols, value):
    def body(i, _):
        for j in range(cols // 16):
            ref[i, pl.ds(j * 16, 16)] = jnp.full((16,), value, jnp.float32)
        return 0
    lax.fori_loop(0, rows, body, 0)


def _sc_counts(cols_ss, cols_tt, cols_st, cols_ts):
    out_t = tuple(jax.ShapeDtypeStruct((N_PAD, 16), jnp.float32)
                  for _ in range(4))

    @functools.partial(
        pl.kernel, out_type=out_t, mesh=_sc_mesh(),
        compiler_params=pltpu.CompilerParams(use_tc_tiling_on_sc=False),
        scratch_types=[
            pltpu.VMEM((NEB, EB), jnp.int32),
            pltpu.VMEM((EB, 16), jnp.float32),
            pltpu.VMEM_SHARED((N_PAD, 16), jnp.float32),
            pltpu.VMEM_SHARED((N_PAD, 16), jnp.float32),
            pltpu.VMEM((ZB, 16), jnp.float32),
        ],
    )
    def k(c_ss, c_tt, c_st, c_ts, o_ss, o_tt, o_st, o_ts,
          idx_v, ones_v, acc_a, acc_b, zbuf):
        c = lax.axis_index("c")
        s = lax.axis_index("s")
        _fill2d(ones_v, EB, 16, 1.0)
        _fill2d(zbuf, ZB, 16, 0.0)
        def zero(i, _):
            pltpu.sync_copy(zbuf, acc_a.at[pl.ds(s * RPT + i * ZB, ZB)])
            pltpu.sync_copy(zbuf, acc_b.at[pl.ds(s * RPT + i * ZB, ZB)])
            return 0
        lax.fori_loop(0, RPT // ZB, zero, 0)
        plsc.subcore_barrier()

        for cc, (ca, cb) in enumerate(((c_ss, c_ts), (c_tt, c_st))):
            @pl.when(c == cc)
            def _():
                for src_h, acc in ((ca, acc_a), (cb, acc_b)):
                    pltpu.sync_copy(src_h.at[s], idx_v)
                    def body(j, _):
                        pltpu.sync_copy(ones_v, acc.at[idx_v.at[j]], add=True)
                        return 0
                    lax.fori_loop(0, NEB, body, 0)
        plsc.subcore_barrier()
        for cc, (oa, ob) in enumerate(((o_ss, o_ts), (o_tt, o_st))):
            @pl.when(c == cc)
            def _():
                pltpu.sync_copy(acc_a.at[pl.ds(s * RPT, RPT)],
                                oa.at[pl.ds(s * RPT, RPT)])
                pltpu.sync_copy(acc_b.at[pl.ds(s * RPT, RPT)],
                                ob.at[pl.ds(s * RPT, RPT)])

    return k(cols_ss, cols_tt, cols_st, cols_ts)


def _sc_pair_sums(tbl_a, tbl_b, zrows, ra, ca, rb, cb):
    out_t = tuple(jax.ShapeDtypeStruct((N_PAD, H), jnp.float32)
                  for _ in range(2))

    @functools.partial(
        pl.kernel, out_type=out_t, mesh=_sc_mesh(),
        compiler_params=pltpu.CompilerParams(use_tc_tiling_on_sc=False),
        scratch_types=[
            pltpu.VMEM((NEB, EB), jnp.int32),
            pltpu.VMEM((NEB, EB), jnp.int32),
            pltpu.VMEM((EB, 32), jnp.float32),
            pltpu.VMEM((EB, 32), jnp.float32),
            pltpu.VMEM((EB, 32), jnp.float32),
            pltpu.VMEM((EB, 32), jnp.float32),
            pltpu.VMEM((EB, 32), jnp.float32),
            pltpu.VMEM_SHARED((N_PAD, 32), jnp.float32),
            pltpu.SemaphoreType.DMA,
            pltpu.SemaphoreType.DMA,
        ],
    )
    def k(ta_h, tb_h, z_h,
          r_a, c_a, r_b, c_b,
          o_a, o_b,
          row_v, col_v, g0, g1, g2, g3, g4, acc, gsem, ssem):
        c = lax.axis_index("c")
        s = lax.axis_index("s")
        gbufs = (g0, g1, g2, g3, g4)
        cfg = ((ta_h, r_a, c_a, o_a),
               (tb_h, r_b, c_b, o_b))

        def add_inplace(mul, off):
            def body(i, _):
                for u in range(EB // 16):
                    sl = (i, pl.ds(16 * u, 16))
                    row_v[sl] = row_v[sl] * mul + off
                return 0
            lax.fori_loop(0, NEB, body, 0)

        for cc in range(NC):
            @pl.when(c == cc)
            def _():
                for tbl, r_h, c_h, o_h in cfg:
                    pltpu.sync_copy(r_h.at[s], row_v)
                    pltpu.sync_copy(c_h.at[s], col_v)
                    add_inplace(4, 2 * cc)
                    for q, kk in enumerate((2 * cc, 2 * cc + 1)):
                        if q == 1:
                            add_inplace(1, 1)
                        pltpu.sync_copy(z_h, acc.at[pl.ds(s * RPT, RPT)])
                        plsc.subcore_barrier()
                        pltpu.async_copy(tbl.at[row_v.at[0]], g0, gsem)
                        pltpu.async_copy(tbl.at[row_v.at[1]], g1, gsem)
                        pltpu.async_copy(tbl.at[row_v.at[2]], g2, gsem)
                        pltpu.async_copy(tbl.at[row_v.at[3]], g3, gsem)

                        def body(j, _):
                            for par in range(5):
                                g_cur = gbufs[par]
                                g_pre = gbufs[(par - 1) % 5]
                                g_nxt = gbufs[(par + 4) % 5]

                                @pl.when(lax.rem(j, 5) == par)
                                def _():
                                    pltpu.make_async_copy(
                                        tbl.at[row_v.at[j]], g_cur,
                                        gsem).wait()

                                    @pl.when(j >= 1)
                                    def _():
                                        pltpu.make_async_copy(
                                            g_pre,
                                            acc.at[col_v.at[j - 1]],
                                            ssem).wait()
                                    pltpu.async_copy(
                                        g_cur, acc.at[col_v.at[j]],
                                        ssem, add=True)

                                    @pl.when(j + 4 < NEB)
                                    def _():
                                        pltpu.async_copy(
                                            tbl.at[row_v.at[j + 4]],
                                            g_nxt, gsem)
                            return 0
                        lax.fori_loop(0, NEB, body, 0)
                        pltpu.make_async_copy(
                            gbufs[(NEB - 1) % 5],
                            acc.at[col_v.at[NEB - 1]], ssem).wait()
                        plsc.subcore_barrier()
                        pltpu.sync_copy(
                            acc.at[pl.ds(s * RPT, RPT)],
                            o_h.at[pl.ds(s * RPT, RPT),
                                   pl.ds(32 * kk, 32)])
                        plsc.subcore_barrier()

    return k(tbl_a, tbl_b, zrows, ra, ca, rb, cb)


def _sc_edge_gather(x3, idx_w):
    out_t = jax.ShapeDtypeStruct((N_EDGES, H), jnp.float32)

    @functools.partial(
        pl.kernel, out_type=out_t, mesh=_sc_mesh(),
        compiler_params=pltpu.CompilerParams(use_tc_tiling_on_sc=False),
        scratch_types=[
            pltpu.VMEM((NEBW, EBW), jnp.int32),
            pltpu.VMEM((EBW, H), jnp.float32),
            pltpu.VMEM((EBW, H), jnp.float32),
            pltpu.SemaphoreType.DMA,
        ],
    )
    def k(tbl, i_h, o_h, idx_v, gbuf_a, gbuf_b, sem):
        c = lax.axis_index("c")
        s = lax.axis_index("s")
        wid = s * NC + c
        base = wid * EPW
        pltpu.sync_copy(i_h.at[wid], idx_v)
        pltpu.async_copy(tbl.at[idx_v.at[0]], gbuf_a, sem)
        def body(j, _):
            for par, (g_cur, g_nxt) in enumerate(
                    ((gbuf_a, gbuf_b), (gbuf_b, gbuf_a))):
                @pl.when(lax.rem(j, 2) == par)
                def _():
                    pltpu.make_async_copy(
                        tbl.at[idx_v.at[j]], g_cur, sem).wait()
                    @pl.when(j < NEBW - 1)
                    def _():
                        pltpu.async_copy(
                            tbl.at[idx_v.at[j + 1]], g_nxt, sem)
                    pltpu.sync_copy(
                        g_cur, o_h.at[pl.ds(base + j * EBW, EBW)])
            return 0
        lax.fori_loop(0, NEBW, body, 0)

    return k(x3, idx_w)


BN = 2000
BE = 2000


def _dot(a, b):
    return jax.lax.dot_general(a, b, (((1,), (0,)), ((), ())),
                               preferred_element_type=jnp.float32)


def _enc_body(x_ref, w1_ref, b1_ref, w2_ref, b2_ref, of_ref):
    h = jnp.maximum(_dot(x_ref[...], w1_ref[...]) + b1_ref[...], 0.0)
    of_ref[...] = _dot(h, w2_ref[...]) + b2_ref[...]


def _encoder(x, w1t, b1, w2t, b2):
    n = x.shape[0]
    grid = (n // BN,)
    wspec = pl.BlockSpec((H, H), lambda i: (0, 0))
    bspec = pl.BlockSpec((1, H), lambda i: (0, 0))
    return pl.pallas_call(
        _enc_body,
        grid=grid,
        in_specs=[pl.BlockSpec((BN, H), lambda i: (i, 0)),
                  wspec, bspec, wspec, bspec],
        out_specs=pl.BlockSpec((BN, H), lambda i: (i, 0)),
        out_shape=jax.ShapeDtypeStruct((n, H), jnp.float32),
    )(x, w1t, b1, w2t, b2)


def _comb_body(sa_ref, ca_ref, sb_ref, cb_ref, x_ref,
               wa_ref, wb_ref, wc_ref, b_ref, o_ref, *, residual):
    inva = 1.0 / jnp.maximum(ca_ref[...][:, 0:1], 1.0)
    invb = 1.0 / jnp.maximum(cb_ref[...][:, 0:1], 1.0)
    acc = (_dot(sa_ref[...], wa_ref[...]) * inva
           + _dot(sb_ref[...], wb_ref[...]) * invb
           + _dot(x_ref[...], wc_ref[...]))
    o = 0.5 * acc + b_ref[...]
    if residual:
        o = o + x_ref[...]
    o_ref[...] = jnp.maximum(o, 0.0)


def _combine(sums_a, cnt_a, sums_b, cnt_b, x, wa, wb, wc, b, residual):
    n = x.shape[0]
    grid = (n // BN,)
    nspec = pl.BlockSpec((BN, H), lambda i: (i, 0))
    cntspec = pl.BlockSpec((BN, 16), lambda i: (i, 0))
    wspec = pl.BlockSpec((H, H), lambda i: (0, 0))
    bspec = pl.BlockSpec((1, H), lambda i: (0, 0))
    return pl.pallas_call(
        functools.partial(_comb_body, residual=residual),
        grid=grid,
        in_specs=[nspec, cntspec, nspec, cntspec, nspec,
                  wspec, wspec, wspec, bspec],
        out_specs=nspec,
        out_shape=jax.ShapeDtypeStruct((n, H), jnp.float32),
    )(sums_a, cnt_a, sums_b, cnt_b, x, wa, wb, wc, b)


def _readout_body(sh_ref, th_ref, ea_ref,
                  g1s_ref, g1t_ref, gb1_ref, g2_ref, gb2_ref,
                  w1a_ref, w1b_ref, eb1_ref, w2_ref, eb2_ref,
                  w3_ref, eb3_ref, o_ref):
    sh = sh_ref[...]
    th = th_ref[...]
    h1 = jnp.maximum(_dot(sh, g1s_ref[...]) + _dot(th, g1t_ref[...])
                     + gb1_ref[...], 0.0)
    gate = jax.nn.sigmoid(_dot(h1, g2_ref[...]) + gb2_ref[...])
    comb = sh * gate + th * (1.0 - gate)
    h2 = jnp.maximum(_dot(comb, w1a_ref[...]) + _dot(ea_ref[...], w1b_ref[...])
                     + eb1_ref[...], 0.0)
    h3 = jnp.maximum(_dot(h2, w2_ref[...]) + eb2_ref[...], 0.0)
    o_ref[...] = _dot(h3, w3_ref[...]) + eb3_ref[...]


def _readout(src_h, tgt_h, edge_attr, g1s, g1t, gb1, g2, gb2,
             w1a, w1b, eb1, w2, eb2, w3, eb3):
    grid = (N_EDGES // BE,)
    espec = pl.BlockSpec((BE, H), lambda i: (i, 0))

    def c(shape):
        return pl.BlockSpec(shape, lambda i: (0, 0))

    return pl.pallas_call(
        _readout_body,
        grid=grid,
        in_specs=[espec, espec, pl.BlockSpec((BE, ED), lambda i: (i, 0)),
                  c((H, H)), c((H, H)), c((1, H)), c((H, 1)), c((1, 1)),
                  c((H, H)), c((ED, H)), c((1, H)), c((H, H // 2)),
                  c((1, H // 2)), c((H // 2, OUT)), c((1, OUT))],
        out_specs=pl.BlockSpec((BE, OUT), lambda i: (i, 0)),
        out_shape=jax.ShapeDtypeStruct((N_EDGES, OUT), jnp.float32),
    )(src_h, tgt_h, edge_attr, g1s, g1t, gb1, g2, gb2,
      w1a, w1b, eb1, w2, eb2, w3, eb3)


def kernel(x_source, x_target, edge_attr_ss, edge_attr_tt, edge_attr_st,
           edge_attr_ts, src_W1, src_b1, src_W2, src_b2, tgt_W1, tgt_b1,
           tgt_W2, tgt_b2, conv_Wl, conv_bl, conv_Wr, gate_W1, gate_b1,
           gate_W2, gate_b2, emlp_W1, emlp_b1, emlp_W2, emlp_b2, emlp_W3,
           emlp_b3, edge_index_ss, edge_index_tt, edge_index_st,
           edge_index_ts):
    ei = {"ss": edge_index_ss, "tt": edge_index_tt,
          "st": edge_index_st, "ts": edge_index_ts}
    ridx = {k: v[0].astype(jnp.int32).reshape(NT, NEB, EB)
            for k, v in ei.items()}
    cidx = {k: v[1].astype(jnp.int32).reshape(NT, NEB, EB)
            for k, v in ei.items()}
    zrows = jnp.zeros((RPT, 32), jnp.float32)

    cnt_ss, cnt_tt, cnt_st, cnt_ts = _sc_counts(
        cidx["ss"], cidx["tt"], cidx["st"], cidx["ts"])

    hs = _encoder(x_source, src_W1.T, src_b1.reshape(1, H),
                  src_W2.T, src_b2.reshape(1, H))
    ht = _encoder(x_target, tgt_W1.T, tgt_b1.reshape(1, H),
                  tgt_W2.T, tgt_b2.reshape(1, H))

    def layer(l, xs, xt, residual):
        xs_tbl = xs.reshape(4 * N_NODES, 32)
        xt_tbl = xt.reshape(4 * N_NODES, 32)
        s_ss, s_ts = _sc_pair_sums(xs_tbl, xt_tbl, zrows,
                                   ridx["ss"], cidx["ss"],
                                   ridx["ts"], cidx["ts"])
        s_tt, s_st = _sc_pair_sums(xt_tbl, xs_tbl, zrows,
                                   ridx["tt"], cidx["tt"],
                                   ridx["st"], cidx["st"])
        wl = conv_Wl[l]
        wr = conv_Wr[l]
        bl = conv_bl[l]
        o_s = _combine(s_ss, cnt_ss, s_ts, cnt_ts, xs,
                       wl[0].T, wl[3].T, (wr[0] + wr[3]).T,
                       (0.5 * (bl[0] + bl[3])).reshape(1, H), residual)
        o_t = _combine(s_tt, cnt_tt, s_st, cnt_st, xt,
                       wl[1].T, wl[2].T, (wr[1] + wr[2]).T,
                       (0.5 * (bl[1] + bl[2])).reshape(1, H), residual)
        return o_s, o_t

    x1s, x1t = layer(0, hs, ht, residual=False)
    x2s, x2t = layer(1, x1s, x1t, residual=True)
    x3s, x3t = layer(2, x2s, x2t, residual=True)

    rows_w = edge_index_st[0].astype(jnp.int32).reshape(NW, NEBW, EBW)
    cols_w = edge_index_st[1].astype(jnp.int32).reshape(NW, NEBW, EBW)
    src_h = _sc_edge_gather(x3s, rows_w)
    tgt_h = _sc_edge_gather(x3t, cols_w)

    return _readout(
        src_h, tgt_h, edge_attr_st,
        gate_W1[:, :H].T, gate_W1[:, H:].T, gate_b1.reshape(1, H),
        gate_W2.T, gate_b2.reshape(1, 1),
        emlp_W1[:, :H].T, emlp_W1[:, H:].T, emlp_b1.reshape(1, H),
        emlp_W2.T, emlp_b2.reshape(1, H // 2),
        emlp_W3.T, emlp_b3.reshape(1, OUT))


# speedup vs baseline: 4.3602x; 1.0234x over previous
import functools

import jax
import jax.numpy as jnp
from jax import lax
from jax.experimental import pallas as pl
from jax.experimental.pallas import tpu as pltpu
from jax.experimental.pallas import tpu_sc as plsc

N_NODES = 50000
N_EDGES = 128000
H = 128
ED = 16
OUT = 2

NC = 2
NT = 16
NW = NC * NT

EPT = N_EDGES // NT
EB = 80
NEB = EPT // EB
EPW = N_EDGES // NW
EBW = 80
NEBW = EPW // EBW
N_PAD = 50048
RPT = N_PAD // NT
ZB = 184


def _sc_mesh():
    return plsc.VectorSubcoreMesh(core_axis_name="c", subcore_axis_name="s",
                                  num_cores=NC, num_subcores=NT)


def _fill2d(ref, rows, cols, value):
    def body(i, _):
        for j in range(cols // 16):
            ref[i, pl.ds(j * 16, 16)] = jnp.full((16,), value, jnp.float32)
        return 0
    lax.fori_loop(0, rows, body, 0)


def _sc_counts(cols_ss, cols_tt, cols_st, cols_ts):
    out_t = tuple(jax.ShapeDtypeStruct((N_PAD, 16), jnp.float32)
                  for _ in range(4))

    @functools.partial(
        pl.kernel, out_type=out_t, mesh=_sc_mesh(),
        compiler_params=pltpu.CompilerParams(use_tc_tiling_on_sc=False),
        scratch_types=[
            pltpu.VMEM((NEB, EB), jnp.int32),
            pltpu.VMEM((EB, 16), jnp.float32),
            pltpu.VMEM_SHARED((N_PAD, 16), jnp.float32),
            pltpu.VMEM_SHARED((N_PAD, 16), jnp.float32),
            pltpu.VMEM((ZB, 16), jnp.float32),
        ],
    )
    def k(c_ss, c_tt, c_st, c_ts, o_ss, o_tt, o_st, o_ts,
          idx_v, ones_v, acc_a, acc_b, zbuf):
        c = lax.axis_index("c")
        s = lax.axis_index("s")
        _fill2d(ones_v, EB, 16, 1.0)
        _fill2d(zbuf, ZB, 16, 0.0)
        def zero(i, _):
            pltpu.sync_copy(zbuf, acc_a.at[pl.ds(s * RPT + i * ZB, ZB)])
            pltpu.sync_copy(zbuf, acc_b.at[pl.ds(s * RPT + i * ZB, ZB)])
            return 0
        lax.fori_loop(0, RPT // ZB, zero, 0)
        plsc.subcore_barrier()

        for cc, (ca, cb) in enumerate(((c_ss, c_ts), (c_tt, c_st))):
            @pl.when(c == cc)
            def _():
                for src_h, acc in ((ca, acc_a), (cb, acc_b)):
                    pltpu.sync_copy(src_h.at[s], idx_v)
                    def body(j, _):
                        pltpu.sync_copy(ones_v, acc.at[idx_v.at[j]], add=True)
                        return 0
                    lax.fori_loop(0, NEB, body, 0)
        plsc.subcore_barrier()
        for cc, (oa, ob) in enumerate(((o_ss, o_ts), (o_tt, o_st))):
            @pl.when(c == cc)
            def _():
                pltpu.sync_copy(acc_a.at[pl.ds(s * RPT, RPT)],
                                oa.at[pl.ds(s * RPT, RPT)])
                pltpu.sync_copy(acc_b.at[pl.ds(s * RPT, RPT)],
                                ob.at[pl.ds(s * RPT, RPT)])

    return k(cols_ss, cols_tt, cols_st, cols_ts)


def _sc_pair_sums(tbl_a, tbl_b, zrows, ra, ca, rb, cb):
    out_t = tuple(jax.ShapeDtypeStruct((N_PAD, H), jnp.float32)
                  for _ in range(2))

    @functools.partial(
        pl.kernel, out_type=out_t, mesh=_sc_mesh(),
        compiler_params=pltpu.CompilerParams(use_tc_tiling_on_sc=False),
        scratch_types=[
            pltpu.VMEM((NEB, EB), jnp.int32),
            pltpu.VMEM((NEB, EB), jnp.int32),
            pltpu.VMEM((EB, 32), jnp.float32),
            pltpu.VMEM((EB, 32), jnp.float32),
            pltpu.VMEM((EB, 32), jnp.float32),
            pltpu.VMEM((EB, 32), jnp.float32),
            pltpu.VMEM((EB, 32), jnp.float32),
            pltpu.VMEM_SHARED((N_PAD, 32), jnp.float32),
            pltpu.SemaphoreType.DMA,
            pltpu.SemaphoreType.DMA,
        ],
    )
    def k(ta_h, tb_h, z_h,
          r_a, c_a, r_b, c_b,
          o_a, o_b,
          row_v, col_v, g0, g1, g2, g3, g4, acc, gsem, ssem):
        c = lax.axis_index("c")
        s = lax.axis_index("s")
        gbufs = (g0, g1, g2, g3, g4)
        cfg = ((ta_h, r_a, c_a, o_a),
               (tb_h, r_b, c_b, o_b))

        def add_inplace(mul, off):
            def body(i, _):
                for u in range(EB // 16):
                    sl = (i, pl.ds(16 * u, 16))
                    row_v[sl] = row_v[sl] * mul + off
                return 0
            lax.fori_loop(0, NEB, body, 0)

        for cc in range(NC):
            @pl.when(c == cc)
            def _():
                for tbl, r_h, c_h, o_h in cfg:
                    pltpu.sync_copy(r_h.at[s], row_v)
                    pltpu.sync_copy(c_h.at[s], col_v)
                    add_inplace(4, 2 * cc)
                    for q, kk in enumerate((2 * cc, 2 * cc + 1)):
                        if q == 1:
                            add_inplace(1, 1)
                        pltpu.sync_copy(z_h, acc.at[pl.ds(s * RPT, RPT)])
                        plsc.subcore_barrier()
                        pltpu.async_copy(tbl.at[row_v.at[0]], g0, gsem)
                        pltpu.async_copy(tbl.at[row_v.at[1]], g1, gsem)
                        pltpu.async_copy(tbl.at[row_v.at[2]], g2, gsem)
                        pltpu.async_copy(tbl.at[row_v.at[3]], g3, gsem)

                        def body(j, _):
                            for par in range(5):
                                g_cur = gbufs[par]
                                g_pre = gbufs[(par - 1) % 5]
                                g_nxt = gbufs[(par + 4) % 5]

                                @pl.when(lax.rem(j, 5) == par)
                                def _():
                                    pltpu.make_async_copy(
                                        tbl.at[row_v.at[j]], g_cur,
                                        gsem).wait()

                                    @pl.when(j >= 1)
                                    def _():
                                        pltpu.make_async_copy(
                                            g_pre,
                                            acc.at[col_v.at[j - 1]],
                                            ssem).wait()
                                    pltpu.async_copy(
                                        g_cur, acc.at[col_v.at[j]],
                                        ssem, add=True)

                                    @pl.when(j + 4 < NEB)
                                    def _():
                                        pltpu.async_copy(
                                            tbl.at[row_v.at[j + 4]],
                                            g_nxt, gsem)
                            return 0
                        lax.fori_loop(0, NEB, body, 0)
                        pltpu.make_async_copy(
                            gbufs[(NEB - 1) % 5],
                            acc.at[col_v.at[NEB - 1]], ssem).wait()
                        plsc.subcore_barrier()
                        pltpu.sync_copy(
                            acc.at[pl.ds(s * RPT, RPT)],
                            o_h.at[pl.ds(s * RPT, RPT),
                                   pl.ds(32 * kk, 32)])
                        plsc.subcore_barrier()

    return k(tbl_a, tbl_b, zrows, ra, ca, rb, cb)


def _sc_edge_gather(x3, idx_w):
    out_t = jax.ShapeDtypeStruct((N_EDGES, H), jnp.float32)

    @functools.partial(
        pl.kernel, out_type=out_t, mesh=_sc_mesh(),
        compiler_params=pltpu.CompilerParams(use_tc_tiling_on_sc=False),
        scratch_types=[
            pltpu.VMEM((NEBW, EBW), jnp.int32),
            pltpu.VMEM((EBW, H), jnp.float32),
            pltpu.VMEM((EBW, H), jnp.float32),
            pltpu.SemaphoreType.DMA,
        ],
    )
    def k(tbl, i_h, o_h, idx_v, gbuf_a, gbuf_b, sem):
        c = lax.axis_index("c")
        s = lax.axis_index("s")
        wid = s * NC + c
        base = wid * EPW
        pltpu.sync_copy(i_h.at[wid], idx_v)
        pltpu.async_copy(tbl.at[idx_v.at[0]], gbuf_a, sem)
        def body(j, _):
            for par, (g_cur, g_nxt) in enumerate(
                    ((gbuf_a, gbuf_b), (gbuf_b, gbuf_a))):
                @pl.when(lax.rem(j, 2) == par)
                def _():
                    pltpu.make_async_copy(
                        tbl.at[idx_v.at[j]], g_cur, sem).wait()
                    @pl.when(j < NEBW - 1)
                    def _():
                        pltpu.async_copy(
                            tbl.at[idx_v.at[j + 1]], g_nxt, sem)
                    pltpu.sync_copy(
                        g_cur, o_h.at[pl.ds(base + j * EBW, EBW)])
            return 0
        lax.fori_loop(0, NEBW, body, 0)

    return k(x3, idx_w)


BN = 5000
BE = 4000


def _dot(a, b):
    return jax.lax.dot_general(a, b, (((1,), (0,)), ((), ())),
                               preferred_element_type=jnp.float32)


def _enc_body(x_ref, w1_ref, b1_ref, w2_ref, b2_ref, of_ref):
    h = jnp.maximum(_dot(x_ref[...], w1_ref[...]) + b1_ref[...], 0.0)
    of_ref[...] = _dot(h, w2_ref[...]) + b2_ref[...]


def _encoder(x, w1t, b1, w2t, b2):
    n = x.shape[0]
    grid = (n // BN,)
    wspec = pl.BlockSpec((H, H), lambda i: (0, 0))
    bspec = pl.BlockSpec((1, H), lambda i: (0, 0))
    return pl.pallas_call(
        _enc_body,
        grid=grid,
        in_specs=[pl.BlockSpec((BN, H), lambda i: (i, 0)),
                  wspec, bspec, wspec, bspec],
        out_specs=pl.BlockSpec((BN, H), lambda i: (i, 0)),
        out_shape=jax.ShapeDtypeStruct((n, H), jnp.float32),
    )(x, w1t, b1, w2t, b2)


def _comb_body(sa_ref, ca_ref, sb_ref, cb_ref, x_ref,
               wa_ref, wb_ref, wc_ref, b_ref, o_ref, *, residual):
    inva = 1.0 / jnp.maximum(ca_ref[...][:, 0:1], 1.0)
    invb = 1.0 / jnp.maximum(cb_ref[...][:, 0:1], 1.0)
    acc = (_dot(sa_ref[...], wa_ref[...]) * inva
           + _dot(sb_ref[...], wb_ref[...]) * invb
           + _dot(x_ref[...], wc_ref[...]))
    o = 0.5 * acc + b_ref[...]
    if residual:
        o = o + x_ref[...]
    o_ref[...] = jnp.maximum(o, 0.0)


def _combine(sums_a, cnt_a, sums_b, cnt_b, x, wa, wb, wc, b, residual):
    n = x.shape[0]
    grid = (n // BN,)
    nspec = pl.BlockSpec((BN, H), lambda i: (i, 0))
    cntspec = pl.BlockSpec((BN, 16), lambda i: (i, 0))
    wspec = pl.BlockSpec((H, H), lambda i: (0, 0))
    bspec = pl.BlockSpec((1, H), lambda i: (0, 0))
    return pl.pallas_call(
        functools.partial(_comb_body, residual=residual),
        grid=grid,
        in_specs=[nspec, cntspec, nspec, cntspec, nspec,
                  wspec, wspec, wspec, bspec],
        out_specs=nspec,
        out_shape=jax.ShapeDtypeStruct((n, H), jnp.float32),
    )(sums_a, cnt_a, sums_b, cnt_b, x, wa, wb, wc, b)


def _readout_body(sh_ref, th_ref, ea_ref,
                  g1s_ref, g1t_ref, gb1_ref, g2_ref, gb2_ref,
                  w1a_ref, w1b_ref, eb1_ref, w2_ref, eb2_ref,
                  w3_ref, eb3_ref, o_ref):
    sh = sh_ref[...]
    th = th_ref[...]
    h1 = jnp.maximum(_dot(sh, g1s_ref[...]) + _dot(th, g1t_ref[...])
                     + gb1_ref[...], 0.0)
    gate = jax.nn.sigmoid(_dot(h1, g2_ref[...]) + gb2_ref[...])
    comb = sh * gate + th * (1.0 - gate)
    h2 = jnp.maximum(_dot(comb, w1a_ref[...]) + _dot(ea_ref[...], w1b_ref[...])
                     + eb1_ref[...], 0.0)
    h3 = jnp.maximum(_dot(h2, w2_ref[...]) + eb2_ref[...], 0.0)
    o_ref[...] = _dot(h3, w3_ref[...]) + eb3_ref[...]


def _readout(src_h, tgt_h, edge_attr, g1s, g1t, gb1, g2, gb2,
             w1a, w1b, eb1, w2, eb2, w3, eb3):
    grid = (N_EDGES // BE,)
    espec = pl.BlockSpec((BE, H), lambda i: (i, 0))

    def c(shape):
        return pl.BlockSpec(shape, lambda i: (0, 0))

    return pl.pallas_call(
        _readout_body,
        grid=grid,
        in_specs=[espec, espec, pl.BlockSpec((BE, ED), lambda i: (i, 0)),
                  c((H, H)), c((H, H)), c((1, H)), c((H, 1)), c((1, 1)),
                  c((H, H)), c((ED, H)), c((1, H)), c((H, H // 2)),
                  c((1, H // 2)), c((H // 2, OUT)), c((1, OUT))],
        out_specs=pl.BlockSpec((BE, OUT), lambda i: (i, 0)),
        out_shape=jax.ShapeDtypeStruct((N_EDGES, OUT), jnp.float32),
    )(src_h, tgt_h, edge_attr, g1s, g1t, gb1, g2, gb2,
      w1a, w1b, eb1, w2, eb2, w3, eb3)


def kernel(x_source, x_target, edge_attr_ss, edge_attr_tt, edge_attr_st,
           edge_attr_ts, src_W1, src_b1, src_W2, src_b2, tgt_W1, tgt_b1,
           tgt_W2, tgt_b2, conv_Wl, conv_bl, conv_Wr, gate_W1, gate_b1,
           gate_W2, gate_b2, emlp_W1, emlp_b1, emlp_W2, emlp_b2, emlp_W3,
           emlp_b3, edge_index_ss, edge_index_tt, edge_index_st,
           edge_index_ts):
    ei = {"ss": edge_index_ss, "tt": edge_index_tt,
          "st": edge_index_st, "ts": edge_index_ts}
    ridx = {k: v[0].astype(jnp.int32).reshape(NT, NEB, EB)
            for k, v in ei.items()}
    cidx = {k: v[1].astype(jnp.int32).reshape(NT, NEB, EB)
            for k, v in ei.items()}
    zrows = jnp.zeros((RPT, 32), jnp.float32)

    cnt_ss, cnt_tt, cnt_st, cnt_ts = _sc_counts(
        cidx["ss"], cidx["tt"], cidx["st"], cidx["ts"])

    hs = _encoder(x_source, src_W1.T, src_b1.reshape(1, H),
                  src_W2.T, src_b2.reshape(1, H))
    ht = _encoder(x_target, tgt_W1.T, tgt_b1.reshape(1, H),
                  tgt_W2.T, tgt_b2.reshape(1, H))

    def layer(l, xs, xt, residual):
        xs_tbl = xs.reshape(4 * N_NODES, 32)
        xt_tbl = xt.reshape(4 * N_NODES, 32)
        s_ss, s_ts = _sc_pair_sums(xs_tbl, xt_tbl, zrows,
                                   ridx["ss"], cidx["ss"],
                                   ridx["ts"], cidx["ts"])
        s_tt, s_st = _sc_pair_sums(xt_tbl, xs_tbl, zrows,
                                   ridx["tt"], cidx["tt"],
                                   ridx["st"], cidx["st"])
        wl = conv_Wl[l]
        wr = conv_Wr[l]
        bl = conv_bl[l]
        o_s = _combine(s_ss, cnt_ss, s_ts, cnt_ts, xs,
                       wl[0].T, wl[3].T, (wr[0] + wr[3]).T,
                       (0.5 * (bl[0] + bl[3])).reshape(1, H), residual)
        o_t = _combine(s_tt, cnt_tt, s_st, cnt_st, xt,
                       wl[1].T, wl[2].T, (wr[1] + wr[2]).T,
                       (0.5 * (bl[1] + bl[2])).reshape(1, H), residual)
        return o_s, o_t

    x1s, x1t = layer(0, hs, ht, residual=False)
    x2s, x2t = layer(1, x1s, x1t, residual=True)
    x3s, x3t = layer(2, x2s, x2t, residual=True)

    rows_w = edge_index_st[0].astype(jnp.int32).reshape(NW, NEBW, EBW)
    cols_w = edge_index_st[1].astype(jnp.int32).reshape(NW, NEBW, EBW)
    src_h = _sc_edge_gather(x3s, rows_w)
    tgt_h = _sc_edge_gather(x3t, cols_w)

    return _readout(
        src_h, tgt_h, edge_attr_st,
        gate_W1[:, :H].T, gate_W1[:, H:].T, gate_b1.reshape(1, H),
        gate_W2.T, gate_b2.reshape(1, 1),
        emlp_W1[:, :H].T, emlp_W1[:, H:].T, emlp_b1.reshape(1, H),
        emlp_W2.T, emlp_b2.reshape(1, H // 2),
        emlp_W3.T, emlp_b3.reshape(1, OUT))
